# Initial kernel scaffold; baseline (speedup 1.0000x reference)
import jax
import jax.numpy as jnp
from jax.experimental import pallas as pl


def kernel(x, adj, adj_prue, training, enc_W1, enc_b1, bn1_g, bn1_b, bn1_m, bn1_v, enc_W2, enc_b2, bn2_g, bn2_b, bn2_m, bn2_v, gc1_Wq, gc1_Wk, gc1_Wv, gc1_Ws, gc1_bq, gc1_bk, gc1_bv, gc1_bs, ch_Wq, ch_Wk, ch_Wv, ch_Ws, ch_bq, ch_bk, ch_bv, ch_bs, gc2_Wq, gc2_Wk, gc2_Wv, gc2_Ws, gc2_bq, gc2_bk, gc2_bv, gc2_bs, codebook, dec_W, dec_b, bnd_g, bnd_b, bnd_m, bnd_v, cluster):
    raise NotImplementedError("write your pallas kernel here")


# speedup vs baseline: 1.8997x; 1.8997x over previous
import functools
import math

import jax
import jax.numpy as jnp
from jax.experimental import pallas as pl
from jax.experimental.pallas import tpu as pltpu

N = 4096
E = 65536
AT = 0.5


def _bn_elu(y, g, b, m, v):
    y = g * (y - m) * jax.lax.rsqrt(v + 1e-4) + b
    return jnp.where(y > 0, y, jnp.exp(y) - 1.0)


def _enc_body(x_ref, W1_ref, b1_ref, g1_ref, bb1_ref, m1_ref, v1_ref,
              W2_ref, b2_ref, g2_ref, bb2_ref, m2_ref, v2_ref, out_ref):
    y = jnp.dot(x_ref[...], W1_ref[...], preferred_element_type=jnp.float32)
    y = _bn_elu(y + b1_ref[...], g1_ref[...], bb1_ref[...], m1_ref[...], v1_ref[...])
    z = jnp.dot(y, W2_ref[...], preferred_element_type=jnp.float32)
    z = _bn_elu(z + b2_ref[...], g2_ref[...], bb2_ref[...], m2_ref[...], v2_ref[...])
    out_ref[...] = z


def _encoder(x, W1, b1, g1, bb1, m1, v1, W2, b2, g2, bb2, m2, v2):
    n, d_in = x.shape
    fh1 = W1.shape[1]
    fh2 = W2.shape[1]
    blk = 512
    full = lambda shape: pl.BlockSpec(shape, lambda i: (0,) * len(shape))
    return pl.pallas_call(
        _enc_body,
        grid=(n // blk,),
        in_specs=[
            pl.BlockSpec((blk, d_in), lambda i: (i, 0)),
            full((d_in, fh1)), full((fh1,)), full((fh1,)), full((fh1,)),
            full((fh1,)), full((fh1,)),
            full((fh1, fh2)), full((fh2,)), full((fh2,)), full((fh2,)),
            full((fh2,)), full((fh2,)),
        ],
        out_specs=pl.BlockSpec((blk, fh2), lambda i: (i, 0)),
        out_shape=jax.ShapeDtypeStruct((n, fh2), jnp.float32),
    )(x, W1, b1, g1, bb1, m1, v1, W2, b2, g2, bb2, m2, v2)


def _tconv_parts(x, ei, Wq, bq, Wk, bk, Wv, bv, Ws, bs):
    src, dst = ei[0], ei[1]
    q = x @ Wq + bq
    k = x @ Wk + bk
    v = x @ Wv + bv
    d = q.shape[1]
    a = jnp.sum(q[dst] * k[src], axis=-1) / math.sqrt(float(d))
    ex = jnp.exp(a)
    den = jax.ops.segment_sum(ex, dst, num_segments=N)
    agg = jax.ops.segment_sum(v[src] * ex[:, None], dst, num_segments=N)
    inv = 1.0 / (den + 1e-16)
    hfull = agg * inv[:, None] + x @ Ws + bs
    g = hfull * inv[:, None]
    return hfull, g, ex


def _datt_apply(ei, ex, g):
    src, dst = ei[0], ei[1]
    keep = (src != dst).astype(ex.dtype)
    return jax.ops.segment_sum((ex * keep)[:, None] * g[dst], src, num_segments=N)


def kernel(x, adj, adj_prue, training,
           enc_W1, enc_b1, bn1_g, bn1_b, bn1_m, bn1_v,
           enc_W2, enc_b2, bn2_g, bn2_b, bn2_m, bn2_v,
           gc1_Wq, gc1_Wk, gc1_Wv, gc1_Ws, gc1_bq, gc1_bk, gc1_bv, gc1_bs,
           ch_Wq, ch_Wk, ch_Wv, ch_Ws, ch_bq, ch_bk, ch_bv, ch_bs,
           gc2_Wq, gc2_Wk, gc2_Wv, gc2_Ws, gc2_bq, gc2_bk, gc2_bv, gc2_bs,
           codebook, dec_W, dec_b, bnd_g, bnd_b, bnd_m, bnd_v, cluster):
    feat = _encoder(x, enc_W1, enc_b1, bn1_g, bn1_b, bn1_m, bn1_v,
                    enc_W2, enc_b2, bn2_g, bn2_b, bn2_m, bn2_v)

    gc1 = (gc1_Wq, gc1_bq, gc1_Wk, gc1_bk, gc1_Wv, gc1_bv, gc1_Ws, gc1_bs)
    ch = (ch_Wq, ch_bq, ch_Wk, ch_bk, ch_Wv, ch_bv, ch_Ws, ch_bs)
    gc2 = (gc2_Wq, gc2_bq, gc2_Wk, gc2_bk, gc2_Wv, gc2_bv, gc2_Ws, gc2_bs)

    h1, g1, ex1 = _tconv_parts(feat, adj, *gc1)
    h1p, g1p, ex1p = _tconv_parts(feat, adj_prue, *gc1)
    comb1 = (1.0 - AT) * _datt_apply(adj, ex1, g1) + AT * _datt_apply(adj_prue, ex1p, g1p)
    xh = jax.nn.relu(comb1)

    x1, ga, exa = _tconv_parts(xh, adj, *ch)
    xp, gap, exap = _tconv_parts(x1, adj_prue, *ch)
    xh2 = jax.nn.relu((1.0 - AT) * _datt_apply(adj, exa, ga) + AT * _datt_apply(adj_prue, exap, gap))

    mu, g2, ex2 = _tconv_parts(xh2, adj, *gc2)
    mup, g2p, ex2p = _tconv_parts(xh2, adj_prue, *gc2)
    mu = (1.0 - AT) * _datt_apply(adj, ex2, g2) + AT * _datt_apply(adj_prue, ex2p, g2p)

    dists = (jnp.sum(feat * feat, 1, keepdims=True)
             + jnp.sum(codebook * codebook, 1)[None, :]
             - 2.0 * (feat @ codebook.T))
    idx = jnp.argmin(dists, axis=1)
    quant = jnp.take(codebook, idx, axis=0)

    z = jnp.concatenate([quant, mu], axis=1)
    de = _bn_elu(z @ dec_W + dec_b, bnd_g, bnd_b, bnd_m, bnd_v)
    qd = 1.0 / (1.0 + jnp.sum((z[:, None, :] - cluster) ** 2, axis=2))
    qd = qd / jnp.sum(qd, axis=1, keepdims=True)
    return z, de, qd, feat


# speedup vs baseline: 4.2487x; 2.2365x over previous
import functools
import math

import jax
import jax.numpy as jnp
from jax import lax
from jax.experimental import pallas as pl
from jax.experimental.pallas import tpu as pltpu
from jax.experimental.pallas import tpu_sc as plsc

N = 4096
E = 65536
AT = 0.5
NC = 2
NS = 16
NW = NC * NS
L = 16
EPW = E // NW
CHK = 128
NCHUNK = EPW // CHK
ROWS = N // NS

_MESH = dict(core_axis_name="c", subcore_axis_name="s", num_cores=NC,
             num_subcores=NS)


def _zero_vmem_2d(ref, rows, d):
    zeros = jnp.zeros((L,), jnp.float32)

    def zrow(r, carry):
        for j in range(d // L):
            ref[r, pl.ds(j * L, L)] = zeros
        return carry

    lax.fori_loop(0, rows, zrow, 0)


def _zero_vmem_1d(ref, n):
    zeros = jnp.zeros((L,), jnp.float32)

    def zblk(i, carry):
        ref[pl.ds(i * L, L)] = zeros
        return carry

    lax.fori_loop(0, n // L, zblk, 0)


@functools.partial(jax.jit, static_argnames=("d", "scale"))
def _sc_ab(q, kv, src, dst, *, d, scale):
    def body(q_hbm, kv_hbm, src_hbm, dst_hbm, ex_hbm, den_hbm, agg_hbm,
             src_v, dst_v, qrows, kvrows, srows, exb,
             den_acc, agg_acc, sem1, sem2):
        c = lax.axis_index("c")
        s = lax.axis_index("s")
        wid = c * NS + s
        base = wid * EPW
        inv_sqrt_d = scale
        eidx = jnp.arange(L, dtype=jnp.int32)

        _zero_vmem_2d(srows, CHK, d)
        _zero_vmem_1d(exb, ROWS)
        pltpu.sync_copy(srows, agg_acc.at[pl.ds(s * ROWS, CHK)])
        pltpu.sync_copy(srows, agg_acc.at[pl.ds(s * ROWS + CHK, CHK)])
        pltpu.sync_copy(exb.at[pl.ds(0, ROWS)], den_acc.at[pl.ds(s * ROWS, ROWS)])
        plsc.subcore_barrier()

        def chunk(i, carry):
            start = base + i * CHK
            pltpu.sync_copy(src_hbm.at[pl.ds(start, CHK)], src_v)
            pltpu.sync_copy(dst_hbm.at[pl.ds(start, CHK)], dst_v)
            d1 = pltpu.async_copy(q_hbm.at[dst_v], qrows, sem1)
            d2 = pltpu.async_copy(kv_hbm.at[src_v], kvrows, sem2)
            d1.wait()
            d2.wait()
            for g in range(CHK // L):
                ev = g * L + eidx

                def fstep(f, acc):
                    fv = jnp.zeros((L,), jnp.int32) + f
                    qv = plsc.load_gather(qrows, [ev, fv])
                    kvv = plsc.load_gather(kvrows, [ev, fv])
                    return acc + qv * kvv

                acc = lax.fori_loop(0, d, fstep, jnp.zeros((L,), jnp.float32),
                                    unroll=8)
                exb[pl.ds(i * CHK + g * L, L)] = jnp.exp(acc * inv_sqrt_d)
            pltpu.sync_copy(exb.at[pl.ds(i * CHK, CHK)], den_acc.at[dst_v],
                            add=True)
            def sgrp(g2, carry2):
                exv = exb[pl.ds(i * CHK + g2 * L, L)]
                for el in range(L):
                    sc = exv[el]
                    for j in range(d // L):
                        srows[g2 * L + el, pl.ds(j * L, L)] = (
                            kvrows[g2 * L + el, pl.ds(d + j * L, L)] * sc)
                return carry2

            lax.fori_loop(0, CHK // L, sgrp, 0)
            pltpu.sync_copy(srows, agg_acc.at[dst_v], add=True)
            return carry

        lax.fori_loop(0, NCHUNK, chunk, 0)
        pltpu.sync_copy(exb, ex_hbm.at[pl.ds(base, EPW)])
        plsc.subcore_barrier()
        pltpu.sync_copy(den_acc.at[pl.ds(s * ROWS, ROWS)],
                        den_hbm.at[c, pl.ds(s * ROWS, ROWS)])
        pltpu.sync_copy(agg_acc.at[pl.ds(s * ROWS, ROWS)],
                        agg_hbm.at[c, pl.ds(s * ROWS, ROWS)])

    f = pl.kernel(
        body,
        out_type=[
            jax.ShapeDtypeStruct((E,), jnp.float32),
            jax.ShapeDtypeStruct((NC, N), jnp.float32),
            jax.ShapeDtypeStruct((NC, N, d), jnp.float32),
        ],
        mesh=plsc.VectorSubcoreMesh(**_MESH),
        compiler_params=pltpu.CompilerParams(needs_layout_passes=False),
        scratch_types=[
            pltpu.VMEM((CHK,), jnp.int32),
            pltpu.VMEM((CHK,), jnp.int32),
            pltpu.VMEM((CHK, d), jnp.float32),
            pltpu.VMEM((CHK, 2 * d), jnp.float32),
            pltpu.VMEM((CHK, d), jnp.float32),
            pltpu.VMEM((EPW,), jnp.float32),
            pltpu.VMEM_SHARED((N,), jnp.float32),
            pltpu.VMEM_SHARED((N, d), jnp.float32),
            pltpu.SemaphoreType.DMA,
            pltpu.SemaphoreType.DMA,
        ],
    )
    return f(q, kv, src, dst)


@functools.partial(jax.jit, static_argnames=("d",))
def _sc_c(g, src, dst, ex, *, d):
    def body(g_hbm, src_hbm, dst_hbm, ex_hbm, out_hbm,
             src_v, dst_v, grows, srows, exb, exk, acc, sem1):
        c = lax.axis_index("c")
        s = lax.axis_index("s")
        wid = c * NS + s
        base = wid * EPW

        _zero_vmem_2d(srows, CHK, d)
        pltpu.sync_copy(srows, acc.at[pl.ds(s * ROWS, CHK)])
        pltpu.sync_copy(srows, acc.at[pl.ds(s * ROWS + CHK, CHK)])
        plsc.subcore_barrier()

        def chunk(i, carry):
            start = base + i * CHK
            pltpu.sync_copy(src_hbm.at[pl.ds(start, CHK)], src_v)
            pltpu.sync_copy(dst_hbm.at[pl.ds(start, CHK)], dst_v)
            pltpu.sync_copy(ex_hbm.at[pl.ds(start, CHK)], exb)
            pltpu.async_copy(g_hbm.at[dst_v], grows, sem1).wait()
            for g_ in range(CHK // L):
                sl = pl.ds(g_ * L, L)
                keep = src_v[sl] != dst_v[sl]
                exk[sl] = jnp.where(keep, exb[sl], 0.0)

            def sgrp(g2, carry2):
                exv = exk[pl.ds(g2 * L, L)]
                for el in range(L):
                    sc = exv[el]
                    for j in range(d // L):
                        srows[g2 * L + el, pl.ds(j * L, L)] = (
                            grows[g2 * L + el, pl.ds(j * L, L)] * sc)
                return carry2

            lax.fori_loop(0, CHK // L, sgrp, 0)
            pltpu.sync_copy(srows, acc.at[src_v], add=True)
            return carry

        lax.fori_loop(0, NCHUNK, chunk, 0)
        plsc.subcore_barrier()
        pltpu.sync_copy(acc.at[pl.ds(s * ROWS, ROWS)],
                        out_hbm.at[c, pl.ds(s * ROWS, ROWS)])

    f = pl.kernel(
        body,
        out_type=[jax.ShapeDtypeStruct((NC, N, d), jnp.float32)],
        mesh=plsc.VectorSubcoreMesh(**_MESH),
        compiler_params=pltpu.CompilerParams(needs_layout_passes=False),
        scratch_types=[
            pltpu.VMEM((CHK,), jnp.int32),
            pltpu.VMEM((CHK,), jnp.int32),
            pltpu.VMEM((CHK, d), jnp.float32),
            pltpu.VMEM((CHK, d), jnp.float32),
            pltpu.VMEM((CHK,), jnp.float32),
            pltpu.VMEM((CHK,), jnp.float32),
            pltpu.VMEM_SHARED((N, d), jnp.float32),
            pltpu.SemaphoreType.DMA,
        ],
    )
    return f(g, src, dst, ex)[0]


def _bn_elu(y, g, b, m, v):
    y = g * (y - m) * lax.rsqrt(v + 1e-4) + b
    return jnp.where(y > 0, y, jnp.exp(y) - 1.0)


def _enc_body(x_ref, W1_ref, b1_ref, g1_ref, bb1_ref, m1_ref, v1_ref,
              W2_ref, b2_ref, g2_ref, bb2_ref, m2_ref, v2_ref, out_ref):
    y = jnp.dot(x_ref[...], W1_ref[...], preferred_element_type=jnp.float32)
    y = _bn_elu(y + b1_ref[...], g1_ref[...], bb1_ref[...], m1_ref[...],
                v1_ref[...])
    z = jnp.dot(y, W2_ref[...], preferred_element_type=jnp.float32)
    z = _bn_elu(z + b2_ref[...], g2_ref[...], bb2_ref[...], m2_ref[...],
                v2_ref[...])
    out_ref[...] = z


def _encoder(x, W1, b1, g1, bb1, m1, v1, W2, b2, g2, bb2, m2, v2):
    n, d_in = x.shape
    fh1 = W1.shape[1]
    fh2 = W2.shape[1]
    blk = 512
    full = lambda shape: pl.BlockSpec(shape, lambda i: (0,) * len(shape))
    return pl.pallas_call(
        _enc_body,
        grid=(n // blk,),
        in_specs=[
            pl.BlockSpec((blk, d_in), lambda i: (i, 0)),
            full((d_in, fh1)), full((fh1,)), full((fh1,)), full((fh1,)),
            full((fh1,)), full((fh1,)),
            full((fh1, fh2)), full((fh2,)), full((fh2,)), full((fh2,)),
            full((fh2,)), full((fh2,)),
        ],
        out_specs=pl.BlockSpec((blk, fh2), lambda i: (i, 0)),
        out_shape=jax.ShapeDtypeStruct((n, fh2), jnp.float32),
    )(x, W1, b1, g1, bb1, m1, v1, W2, b2, g2, bb2, m2, v2)


def _qkvs_direct(x, Wcat, bcat, d):
    n, din = x.shape
    blk = 1024

    def body(x_ref, w_ref, b_ref, q_ref, kv_ref, s_ref):
        y = jnp.dot(x_ref[...], w_ref[...],
                    preferred_element_type=jnp.float32) + b_ref[...]
        q_ref[...] = y[:, :d]
        kv_ref[...] = y[:, d:3 * d]
        s_ref[...] = y[:, 3 * d:]

    full = lambda shape: pl.BlockSpec(shape, lambda i: (0,) * len(shape))
    return pl.pallas_call(
        body,
        grid=(n // blk,),
        in_specs=[pl.BlockSpec((blk, din), lambda i: (i, 0)),
                  full((din, 4 * d)), full((4 * d,))],
        out_specs=[pl.BlockSpec((blk, d), lambda i: (i, 0)),
                   pl.BlockSpec((blk, 2 * d), lambda i: (i, 0)),
                   pl.BlockSpec((blk, d), lambda i: (i, 0))],
        out_shape=[jax.ShapeDtypeStruct((n, d), jnp.float32),
                   jax.ShapeDtypeStruct((n, 2 * d), jnp.float32),
                   jax.ShapeDtypeStruct((n, d), jnp.float32)],
    )(x, Wcat, bcat)


def _qkvs_mix(o2a, o2p, Wcat, bcat, d):
    _, n, din = o2a.shape
    blk = 1024

    def body(a_ref, p_ref, w_ref, b_ref, q_ref, kv_ref, s_ref):
        xin = ((1.0 - AT) * (a_ref[0] + a_ref[1])
               + AT * (p_ref[0] + p_ref[1]))
        xin = jnp.maximum(xin, 0.0)
        y = jnp.dot(xin, w_ref[...],
                    preferred_element_type=jnp.float32) + b_ref[...]
        q_ref[...] = y[:, :d]
        kv_ref[...] = y[:, d:3 * d]
        s_ref[...] = y[:, 3 * d:]

    full = lambda shape: pl.BlockSpec(shape, lambda i: (0,) * len(shape))
    return pl.pallas_call(
        body,
        grid=(n // blk,),
        in_specs=[pl.BlockSpec((NC, blk, din), lambda i: (0, i, 0)),
                  pl.BlockSpec((NC, blk, din), lambda i: (0, i, 0)),
                  full((din, 4 * d)), full((4 * d,))],
        out_specs=[pl.BlockSpec((blk, d), lambda i: (i, 0)),
                   pl.BlockSpec((blk, 2 * d), lambda i: (i, 0)),
                   pl.BlockSpec((blk, d), lambda i: (i, 0))],
        out_shape=[jax.ShapeDtypeStruct((n, d), jnp.float32),
                   jax.ShapeDtypeStruct((n, 2 * d), jnp.float32),
                   jax.ShapeDtypeStruct((n, d), jnp.float32)],
    )(o2a, o2p, Wcat, bcat)


def _comb(den, agg, skip):
    _, n, d = agg.shape
    blk = 1024

    def body(den_ref, agg_ref, skip_ref, h_ref, g_ref):
        inv = 1.0 / (den_ref[0, :] + den_ref[1, :] + 1e-16)
        h = (agg_ref[0] + agg_ref[1]) * inv[:, None] + skip_ref[...]
        h_ref[...] = h
        g_ref[...] = h * inv[:, None]

    return pl.pallas_call(
        body,
        grid=(n // blk,),
        in_specs=[pl.BlockSpec((NC, blk), lambda i: (0, i)),
                  pl.BlockSpec((NC, blk, d), lambda i: (0, i, 0)),
                  pl.BlockSpec((blk, d), lambda i: (i, 0))],
        out_specs=[pl.BlockSpec((blk, d), lambda i: (i, 0)),
                   pl.BlockSpec((blk, d), lambda i: (i, 0))],
        out_shape=[jax.ShapeDtypeStruct((n, d), jnp.float32),
                   jax.ShapeDtypeStruct((n, d), jnp.float32)],
    )(den, agg, skip)


def _decoder(feat, codebook, o2d, o2e, dec_W, dec_b, bg, bb, bm, bv, cluster):
    n = feat.shape[0]
    blk = 512
    ncb = codebook.shape[0]
    d_out = dec_W.shape[1]

    def body(f_ref, cb_ref, a_ref, p_ref, w_ref, b_ref, g_ref, bb_ref,
             m_ref, v_ref, cl_ref, z_ref, de_ref, qd_ref):
        f = f_ref[...]
        cb = cb_ref[...]
        d2 = (jnp.sum(f * f, 1, keepdims=True)
              + jnp.sum(cb * cb, 1)[None, :]
              - 2.0 * lax.dot_general(f, cb, (((1,), (1,)), ((), ())),
                                      preferred_element_type=jnp.float32))
        mn = jnp.min(d2, axis=1, keepdims=True)
        iota = lax.broadcasted_iota(jnp.int32, d2.shape, 1)
        cand = jnp.where(d2 <= mn, iota, ncb)
        idx = jnp.min(cand, axis=1, keepdims=True)
        onehot = (iota == idx).astype(jnp.float32)
        quant = jnp.dot(onehot, cb, preferred_element_type=jnp.float32)
        mu = ((1.0 - AT) * (a_ref[0] + a_ref[1])
              + AT * (p_ref[0] + p_ref[1]))[:, :64]
        z = jnp.concatenate([quant, mu], axis=1)
        z_ref[...] = z
        de = jnp.dot(z, w_ref[...], preferred_element_type=jnp.float32)
        de_ref[...] = _bn_elu(de + b_ref[...], g_ref[...], bb_ref[...],
                              m_ref[...], v_ref[...])
        cl = cl_ref[...]
        dq = (jnp.sum(z * z, 1, keepdims=True)
              + jnp.sum(cl * cl, 1)[None, :]
              - 2.0 * lax.dot_general(z, cl, (((1,), (1,)), ((), ())),
                                      preferred_element_type=jnp.float32))
        q0 = 1.0 / (1.0 + dq)
        qd_ref[...] = q0 / jnp.sum(q0, axis=1, keepdims=True)

    full = lambda shape: pl.BlockSpec(shape, lambda i: (0,) * len(shape))
    fh2 = feat.shape[1]
    dmu = o2d.shape[2]
    ncl = cluster.shape[0]
    return pl.pallas_call(
        body,
        grid=(n // blk,),
        in_specs=[pl.BlockSpec((blk, fh2), lambda i: (i, 0)),
                  full((ncb, fh2)),
                  pl.BlockSpec((NC, blk, dmu), lambda i: (0, i, 0)),
                  pl.BlockSpec((NC, blk, dmu), lambda i: (0, i, 0)),
                  full((fh2 + 64, d_out)), full((d_out,)), full((d_out,)),
                  full((d_out,)), full((d_out,)), full((d_out,)),
                  full((ncl, fh2 + 64))],
        out_specs=[pl.BlockSpec((blk, fh2 + 64), lambda i: (i, 0)),
                   pl.BlockSpec((blk, d_out), lambda i: (i, 0)),
                   pl.BlockSpec((blk, ncl), lambda i: (i, 0))],
        out_shape=[jax.ShapeDtypeStruct((n, fh2 + 64), jnp.float32),
                   jax.ShapeDtypeStruct((n, d_out), jnp.float32),
                   jax.ShapeDtypeStruct((n, ncl), jnp.float32)],
    )(feat, codebook, o2d, o2e, dec_W, dec_b, bg, bb, bm, bv, cluster)


def kernel(x, adj, adj_prue, training,
           enc_W1, enc_b1, bn1_g, bn1_b, bn1_m, bn1_v,
           enc_W2, enc_b2, bn2_g, bn2_b, bn2_m, bn2_v,
           gc1_Wq, gc1_Wk, gc1_Wv, gc1_Ws, gc1_bq, gc1_bk, gc1_bv, gc1_bs,
           ch_Wq, ch_Wk, ch_Wv, ch_Ws, ch_bq, ch_bk, ch_bv, ch_bs,
           gc2_Wq, gc2_Wk, gc2_Wv, gc2_Ws, gc2_bq, gc2_bk, gc2_bv, gc2_bs,
           codebook, dec_W, dec_b, bnd_g, bnd_b, bnd_m, bnd_v, cluster):
    srcA, dstA = adj[0], adj[1]
    srcP, dstP = adj_prue[0], adj_prue[1]

    Wc1 = jnp.concatenate([gc1_Wq, gc1_Wk, gc1_Wv, gc1_Ws], axis=1)
    bc1 = jnp.concatenate([gc1_bq, gc1_bk, gc1_bv, gc1_bs])
    Wc2 = jnp.concatenate([ch_Wq, ch_Wk, ch_Wv, ch_Ws], axis=1)
    bc2 = jnp.concatenate([ch_bq, ch_bk, ch_bv, ch_bs])
    _pw = lambda W: jnp.pad(W, ((0, 0), (0, 64)))
    _pb = lambda b: jnp.pad(b, (0, 64))
    Wc3 = jnp.concatenate([_pw(gc2_Wq), _pw(gc2_Wk), _pw(gc2_Wv),
                           _pw(gc2_Ws)], axis=1)
    bc3 = jnp.concatenate([_pb(gc2_bq), _pb(gc2_bk), _pb(gc2_bv),
                           _pb(gc2_bs)])

    feat = _encoder(x, enc_W1, enc_b1, bn1_g, bn1_b, bn1_m, bn1_v,
                    enc_W2, enc_b2, bn2_g, bn2_b, bn2_m, bn2_v)

    _s128 = 1.0 / math.sqrt(128.0)
    _s64 = 1.0 / math.sqrt(64.0)

    q1, kv1, sk1 = _qkvs_direct(feat, Wc1, bc1, 128)
    exA, denA, aggA = _sc_ab(q1, kv1, srcA, dstA, d=128, scale=_s128)
    exP, denP, aggP = _sc_ab(q1, kv1, srcP, dstP, d=128, scale=_s128)
    h1, g1 = _comb(denA, aggA, sk1)
    h1p, g1p = _comb(denP, aggP, sk1)
    o2A = _sc_c(g1, srcA, dstA, exA, d=128)
    o2P = _sc_c(g1p, srcP, dstP, exP, d=128)

    q2, kv2, sk2 = _qkvs_mix(o2A, o2P, Wc2, bc2, 128)
    exB, denB, aggB = _sc_ab(q2, kv2, srcA, dstA, d=128, scale=_s128)
    x1, gB = _comb(denB, aggB, sk2)
    o2B = _sc_c(gB, srcA, dstA, exB, d=128)
    q3, kv3, sk3 = _qkvs_direct(x1, Wc2, bc2, 128)
    exC, denC, aggC = _sc_ab(q3, kv3, srcP, dstP, d=128, scale=_s128)
    xp, gC = _comb(denC, aggC, sk3)
    o2C = _sc_c(gC, srcP, dstP, exC, d=128)

    q4, kv4, sk4 = _qkvs_mix(o2B, o2C, Wc3, bc3, 128)
    exD, denD, aggD = _sc_ab(q4, kv4, srcA, dstA, d=128, scale=_s64)
    exE, denE, aggE = _sc_ab(q4, kv4, srcP, dstP, d=128, scale=_s64)
    muA, gD = _comb(denD, aggD, sk4)
    muP, gE = _comb(denE, aggE, sk4)
    o2D = _sc_c(gD, srcA, dstA, exD, d=128)
    o2E = _sc_c(gE, srcP, dstP, exE, d=128)

    z, de, qd = _decoder(feat, codebook, o2D, o2E, dec_W, dec_b,
                         bnd_g, bnd_b, bnd_m, bnd_v, cluster)
    return z, de, qd, feat


# speedup vs baseline: 5.3504x; 1.2593x over previous
import functools
import math

import jax
import jax.numpy as jnp
from jax import lax
from jax.experimental import pallas as pl
from jax.experimental.pallas import tpu as pltpu
from jax.experimental.pallas import tpu_sc as plsc

N = 4096
E = 65536
AT = 0.5
NC = 2
NS = 16
NW = NC * NS
L = 16
EPW = E // NW
CHK = 64
NCHUNK = EPW // CHK
ROWS = N // NS

_MESH = dict(core_axis_name="c", subcore_axis_name="s", num_cores=NC,
             num_subcores=NS)


def _zero_vmem_2d(ref, rows, d):
    zeros = jnp.zeros((L,), jnp.float32)

    def zrow(r, carry):
        for j in range(d // L):
            ref[r, pl.ds(j * L, L)] = zeros
        return carry

    lax.fori_loop(0, rows, zrow, 0)


def _zero_vmem_1d(ref, n):
    zeros = jnp.zeros((L,), jnp.float32)

    def zblk(i, carry):
        ref[pl.ds(i * L, L)] = zeros
        return carry

    lax.fori_loop(0, n // L, zblk, 0)


@functools.partial(jax.jit, static_argnames=("d", "scale"))
def _sc_ab(q, kv, src, dst, *, d, scale):
    def body(q_hbm, kv_hbm, src_hbm, dst_hbm, ex_hbm, den_hbm, agg_hbm,
             src_v0, src_v1, dst_v0, dst_v1, dst_s0, dst_s1,
             qrows0, qrows1, kvrows0, kvrows1, srows0, srows1, exb,
             den_acc, agg_acc,
             sem_i0, sem_i1, sem_q0, sem_q1, sem_k0, sem_k1,
             sem_d0, sem_d1, sem_a0, sem_a1):
        c = lax.axis_index("c")
        s = lax.axis_index("s")
        wid = c * NS + s
        base = wid * EPW
        eidx = jnp.arange(L, dtype=jnp.int32)

        SRC = (src_v0, src_v1)
        DST = (dst_v0, dst_v1)
        DSC = (dst_s0, dst_s1)
        QR = (qrows0, qrows1)
        KVR = (kvrows0, kvrows1)
        SR = (srows0, srows1)
        SI = (sem_i0, sem_i1)
        SQ = (sem_q0, sem_q1)
        SK = (sem_k0, sem_k1)
        SD = (sem_d0, sem_d1)
        SA = (sem_a0, sem_a1)

        _zero_vmem_2d(srows0, CHK, d)
        _zero_vmem_1d(exb, ROWS)
        for t in range(ROWS // CHK):
            pltpu.sync_copy(srows0, agg_acc.at[pl.ds(s * ROWS + t * CHK, CHK)])
        pltpu.sync_copy(exb.at[pl.ds(0, ROWS)],
                        den_acc.at[pl.ds(s * ROWS, ROWS)])
        plsc.subcore_barrier()

        def idx_issue(i, r):
            st = base + i * CHK
            pltpu.async_copy(src_hbm.at[pl.ds(st, CHK)], SRC[r], SI[r])
            pltpu.async_copy(dst_hbm.at[pl.ds(st, CHK)], DST[r], SI[r])

        def idx_wait(i, r):
            st = base + i * CHK
            pltpu.make_async_copy(src_hbm.at[pl.ds(st, CHK)], SRC[r], SI[r]).wait()
            pltpu.make_async_copy(dst_hbm.at[pl.ds(st, CHK)], DST[r], SI[r]).wait()

        def gat_issue(r):
            pltpu.async_copy(q_hbm.at[DST[r]], QR[r], SQ[r])
            pltpu.async_copy(kv_hbm.at[SRC[r]], KVR[r], SK[r])

        def gat_wait(r):
            pltpu.make_async_copy(q_hbm.at[DST[r]], QR[r], SQ[r]).wait()
            pltpu.make_async_copy(kv_hbm.at[SRC[r]], KVR[r], SK[r]).wait()

        def sca_issue(i, r):
            pltpu.async_copy(exb.at[pl.ds(i * CHK, CHK)],
                             den_acc.at[DSC[r]], SD[r], add=True)
            pltpu.async_copy(SR[r], agg_acc.at[DSC[r]], SA[r], add=True)

        def sca_wait(i, r):
            pltpu.make_async_copy(exb.at[pl.ds(i * CHK, CHK)],
                                  den_acc.at[DSC[r]], SD[r]).wait()
            pltpu.make_async_copy(SR[r], agg_acc.at[DSC[r]], SA[r]).wait()

        def compute(i, r):
            qr, kvr = QR[r], KVR[r]
            for g in range(CHK // L):
                ev = g * L + eidx

                def fstep(f2, accs):
                    a1, a2 = accs
                    f = 2 * f2
                    fv1 = jnp.zeros((L,), jnp.int32) + f
                    fv2 = fv1 + 1
                    a1 = a1 + (plsc.load_gather(qr, [ev, fv1])
                               * plsc.load_gather(kvr, [ev, fv1]))
                    a2 = a2 + (plsc.load_gather(qr, [ev, fv2])
                               * plsc.load_gather(kvr, [ev, fv2]))
                    return (a1, a2)

                z16 = jnp.zeros((L,), jnp.float32)
                a1, a2 = lax.fori_loop(0, d // 2, fstep, (z16, z16), unroll=8)
                exb[pl.ds(i * CHK + g * L, L)] = jnp.exp((a1 + a2) * scale)

        def snap_idx(r):
            for t in range(CHK // L):
                DSC[r][pl.ds(t * L, L)] = DST[r][pl.ds(t * L, L)]

        def scale_rows(i, r):
            kvr, sr = KVR[r], SR[r]

            def sgrp(g2, carry2):
                exv = exb[pl.ds(i * CHK + g2 * L, L)]
                for el in range(L):
                    sc = exv[el]
                    for j in range(d // L):
                        sr[g2 * L + el, pl.ds(j * L, L)] = (
                            kvr[g2 * L + el, pl.ds(d + j * L, L)] * sc)
                return carry2

            lax.fori_loop(0, CHK // L, sgrp, 0)

        def one(i, r, rn):
            @pl.when(i + 1 < NCHUNK)
            def _():
                idx_wait(i + 1, rn)
                gat_issue(rn)
            gat_wait(r)
            snap_idx(r)
            @pl.when(i + 2 < NCHUNK)
            def _():
                idx_issue(i + 2, r)
            compute(i, r)
            @pl.when(i >= 2)
            def _():
                sca_wait(i - 2, r)
            scale_rows(i, r)
            sca_issue(i, r)

        idx_issue(0, 0)
        idx_wait(0, 0)
        gat_issue(0)
        idx_issue(1, 1)

        def pair(j, carry):
            one(2 * j, 0, 1)
            one(2 * j + 1, 1, 0)
            return carry

        lax.fori_loop(0, NCHUNK // 2, pair, 0)
        sca_wait(NCHUNK - 2, 0)
        sca_wait(NCHUNK - 1, 1)
        pltpu.sync_copy(exb, ex_hbm.at[pl.ds(base, EPW)])
        plsc.subcore_barrier()
        pltpu.sync_copy(den_acc.at[pl.ds(s * ROWS, ROWS)],
                        den_hbm.at[c, pl.ds(s * ROWS, ROWS)])
        pltpu.sync_copy(agg_acc.at[pl.ds(s * ROWS, ROWS)],
                        agg_hbm.at[c, pl.ds(s * ROWS, ROWS)])

    f = pl.kernel(
        body,
        out_type=[
            jax.ShapeDtypeStruct((E,), jnp.float32),
            jax.ShapeDtypeStruct((NC, N), jnp.float32),
            jax.ShapeDtypeStruct((NC, N, d), jnp.float32),
        ],
        mesh=plsc.VectorSubcoreMesh(**_MESH),
        compiler_params=pltpu.CompilerParams(needs_layout_passes=False),
        scratch_types=(
            [pltpu.VMEM((CHK,), jnp.int32)] * 6
            + [pltpu.VMEM((CHK, d), jnp.float32)] * 2
            + [pltpu.VMEM((CHK, 2 * d), jnp.float32)] * 2
            + [pltpu.VMEM((CHK, d), jnp.float32)] * 2
            + [pltpu.VMEM((EPW,), jnp.float32)]
            + [pltpu.VMEM_SHARED((N,), jnp.float32),
               pltpu.VMEM_SHARED((N, d), jnp.float32)]
            + [pltpu.SemaphoreType.DMA] * 10
        ),
    )
    return f(q, kv, src, dst)


@functools.partial(jax.jit, static_argnames=("d",))
def _sc_c(g, src, dst, ex, *, d):
    def body(g_hbm, src_hbm, dst_hbm, ex_hbm, out_hbm,
             src_v0, src_v1, dst_v0, dst_v1, src_s0, src_s1,
             grows0, grows1, srows0, srows1, exb0, exb1, exk,
             acc,
             sem_i0, sem_i1, sem_g0, sem_g1, sem_a0, sem_a1):
        c = lax.axis_index("c")
        s = lax.axis_index("s")
        wid = c * NS + s
        base = wid * EPW

        SRC = (src_v0, src_v1)
        DST = (dst_v0, dst_v1)
        SSC = (src_s0, src_s1)
        GR = (grows0, grows1)
        SR = (srows0, srows1)
        EXB = (exb0, exb1)
        SI = (sem_i0, sem_i1)
        SG = (sem_g0, sem_g1)
        SA = (sem_a0, sem_a1)

        _zero_vmem_2d(srows0, CHK, d)
        for t in range(ROWS // CHK):
            pltpu.sync_copy(srows0, acc.at[pl.ds(s * ROWS + t * CHK, CHK)])
        plsc.subcore_barrier()

        def idx_issue(i, r):
            st = base + i * CHK
            pltpu.async_copy(src_hbm.at[pl.ds(st, CHK)], SRC[r], SI[r])
            pltpu.async_copy(dst_hbm.at[pl.ds(st, CHK)], DST[r], SI[r])
            pltpu.async_copy(ex_hbm.at[pl.ds(st, CHK)], EXB[r], SI[r])

        def idx_wait(i, r):
            st = base + i * CHK
            pltpu.make_async_copy(src_hbm.at[pl.ds(st, CHK)], SRC[r], SI[r]).wait()
            pltpu.make_async_copy(dst_hbm.at[pl.ds(st, CHK)], DST[r], SI[r]).wait()
            pltpu.make_async_copy(ex_hbm.at[pl.ds(st, CHK)], EXB[r], SI[r]).wait()

        def gat_issue(r):
            pltpu.async_copy(g_hbm.at[DST[r]], GR[r], SG[r])

        def gat_wait(r):
            pltpu.make_async_copy(g_hbm.at[DST[r]], GR[r], SG[r]).wait()

        def sca_issue(r):
            pltpu.async_copy(SR[r], acc.at[SSC[r]], SA[r], add=True)

        def sca_wait(r):
            pltpu.make_async_copy(SR[r], acc.at[SSC[r]], SA[r]).wait()

        def snap_idx(r):
            for t in range(CHK // L):
                sl = pl.ds(t * L, L)
                SSC[r][sl] = SRC[r][sl]
                keep = SRC[r][sl] != DST[r][sl]
                exk[sl] = jnp.where(keep, EXB[r][sl], 0.0)

        def scale_rows(r):
            def sgrp(g2, carry2):
                exv = exk[pl.ds(g2 * L, L)]
                for el in range(L):
                    sc = exv[el]
                    for j in range(d // L):
                        SR[r][g2 * L + el, pl.ds(j * L, L)] = (
                            GR[r][g2 * L + el, pl.ds(j * L, L)] * sc)
                return carry2

            lax.fori_loop(0, CHK // L, sgrp, 0)

        def one(i, r, rn):
            @pl.when(i + 1 < NCHUNK)
            def _():
                idx_wait(i + 1, rn)
                gat_issue(rn)
            gat_wait(r)
            snap_idx(r)
            @pl.when(i + 2 < NCHUNK)
            def _():
                idx_issue(i + 2, r)
            @pl.when(i >= 2)
            def _():
                sca_wait(r)
            scale_rows(r)
            sca_issue(r)

        idx_issue(0, 0)
        idx_wait(0, 0)
        gat_issue(0)
        idx_issue(1, 1)

        def pair(j, carry):
            one(2 * j, 0, 1)
            one(2 * j + 1, 1, 0)
            return carry

        lax.fori_loop(0, NCHUNK // 2, pair, 0)
        sca_wait(0)
        sca_wait(1)
        plsc.subcore_barrier()
        pltpu.sync_copy(acc.at[pl.ds(s * ROWS, ROWS)],
                        out_hbm.at[c, pl.ds(s * ROWS, ROWS)])

    f = pl.kernel(
        body,
        out_type=[jax.ShapeDtypeStruct((NC, N, d), jnp.float32)],
        mesh=plsc.VectorSubcoreMesh(**_MESH),
        compiler_params=pltpu.CompilerParams(needs_layout_passes=False),
        scratch_types=(
            [pltpu.VMEM((CHK,), jnp.int32)] * 6
            + [pltpu.VMEM((CHK, d), jnp.float32)] * 4
            + [pltpu.VMEM((CHK,), jnp.float32)] * 3
            + [pltpu.VMEM_SHARED((N, d), jnp.float32)]
            + [pltpu.SemaphoreType.DMA] * 6
        ),
    )
    return f(g, src, dst, ex)[0]


def _bn_elu(y, g, b, m, v):
    y = g * (y - m) * lax.rsqrt(v + 1e-4) + b
    return jnp.where(y > 0, y, jnp.exp(y) - 1.0)


def _enc_body(x_ref, W1_ref, b1_ref, g1_ref, bb1_ref, m1_ref, v1_ref,
              W2_ref, b2_ref, g2_ref, bb2_ref, m2_ref, v2_ref, out_ref):
    y = jnp.dot(x_ref[...], W1_ref[...], preferred_element_type=jnp.float32)
    y = _bn_elu(y + b1_ref[...], g1_ref[...], bb1_ref[...], m1_ref[...],
                v1_ref[...])
    z = jnp.dot(y, W2_ref[...], preferred_element_type=jnp.float32)
    z = _bn_elu(z + b2_ref[...], g2_ref[...], bb2_ref[...], m2_ref[...],
                v2_ref[...])
    out_ref[...] = z


def _encoder(x, W1, b1, g1, bb1, m1, v1, W2, b2, g2, bb2, m2, v2):
    n, d_in = x.shape
    fh1 = W1.shape[1]
    fh2 = W2.shape[1]
    blk = 512
    full = lambda shape: pl.BlockSpec(shape, lambda i: (0,) * len(shape))
    return pl.pallas_call(
        _enc_body,
        grid=(n // blk,),
        in_specs=[
            pl.BlockSpec((blk, d_in), lambda i: (i, 0)),
            full((d_in, fh1)), full((fh1,)), full((fh1,)), full((fh1,)),
            full((fh1,)), full((fh1,)),
            full((fh1, fh2)), full((fh2,)), full((fh2,)), full((fh2,)),
            full((fh2,)), full((fh2,)),
        ],
        out_specs=pl.BlockSpec((blk, fh2), lambda i: (i, 0)),
        out_shape=jax.ShapeDtypeStruct((n, fh2), jnp.float32),
    )(x, W1, b1, g1, bb1, m1, v1, W2, b2, g2, bb2, m2, v2)


def _qkvs_direct(x, Wcat, bcat, d):
    n, din = x.shape
    blk = 1024

    def body(x_ref, w_ref, b_ref, q_ref, kv_ref, s_ref):
        y = jnp.dot(x_ref[...], w_ref[...],
                    preferred_element_type=jnp.float32) + b_ref[...]
        q_ref[...] = y[:, :d]
        kv_ref[...] = y[:, d:3 * d]
        s_ref[...] = y[:, 3 * d:]

    full = lambda shape: pl.BlockSpec(shape, lambda i: (0,) * len(shape))
    return pl.pallas_call(
        body,
        grid=(n // blk,),
        in_specs=[pl.BlockSpec((blk, din), lambda i: (i, 0)),
                  full((din, 4 * d)), full((4 * d,))],
        out_specs=[pl.BlockSpec((blk, d), lambda i: (i, 0)),
                   pl.BlockSpec((blk, 2 * d), lambda i: (i, 0)),
                   pl.BlockSpec((blk, d), lambda i: (i, 0))],
        out_shape=[jax.ShapeDtypeStruct((n, d), jnp.float32),
                   jax.ShapeDtypeStruct((n, 2 * d), jnp.float32),
                   jax.ShapeDtypeStruct((n, d), jnp.float32)],
    )(x, Wcat, bcat)


def _qkvs_mix(o2a, o2p, Wcat, bcat, d):
    _, n, din = o2a.shape
    blk = 1024

    def body(a_ref, p_ref, w_ref, b_ref, q_ref, kv_ref, s_ref):
        xin = ((1.0 - AT) * (a_ref[0] + a_ref[1])
               + AT * (p_ref[0] + p_ref[1]))
        xin = jnp.maximum(xin, 0.0)
        y = jnp.dot(xin, w_ref[...],
                    preferred_element_type=jnp.float32) + b_ref[...]
        q_ref[...] = y[:, :d]
        kv_ref[...] = y[:, d:3 * d]
        s_ref[...] = y[:, 3 * d:]

    full = lambda shape: pl.BlockSpec(shape, lambda i: (0,) * len(shape))
    return pl.pallas_call(
        body,
        grid=(n // blk,),
        in_specs=[pl.BlockSpec((NC, blk, din), lambda i: (0, i, 0)),
                  pl.BlockSpec((NC, blk, din), lambda i: (0, i, 0)),
                  full((din, 4 * d)), full((4 * d,))],
        out_specs=[pl.BlockSpec((blk, d), lambda i: (i, 0)),
                   pl.BlockSpec((blk, 2 * d), lambda i: (i, 0)),
                   pl.BlockSpec((blk, d), lambda i: (i, 0))],
        out_shape=[jax.ShapeDtypeStruct((n, d), jnp.float32),
                   jax.ShapeDtypeStruct((n, 2 * d), jnp.float32),
                   jax.ShapeDtypeStruct((n, d), jnp.float32)],
    )(o2a, o2p, Wcat, bcat)


def _comb(den, agg, skip):
    _, n, d = agg.shape
    blk = 1024

    def body(den_ref, agg_ref, skip_ref, h_ref, g_ref):
        inv = 1.0 / (den_ref[0, :] + den_ref[1, :] + 1e-16)
        h = (agg_ref[0] + agg_ref[1]) * inv[:, None] + skip_ref[...]
        h_ref[...] = h
        g_ref[...] = h * inv[:, None]

    return pl.pallas_call(
        body,
        grid=(n // blk,),
        in_specs=[pl.BlockSpec((NC, blk), lambda i: (0, i)),
                  pl.BlockSpec((NC, blk, d), lambda i: (0, i, 0)),
                  pl.BlockSpec((blk, d), lambda i: (i, 0))],
        out_specs=[pl.BlockSpec((blk, d), lambda i: (i, 0)),
                   pl.BlockSpec((blk, d), lambda i: (i, 0))],
        out_shape=[jax.ShapeDtypeStruct((n, d), jnp.float32),
                   jax.ShapeDtypeStruct((n, d), jnp.float32)],
    )(den, agg, skip)


def _decoder(feat, codebook, o2d, o2e, dec_W, dec_b, bg, bb, bm, bv, cluster):
    n = feat.shape[0]
    blk = 512
    ncb = codebook.shape[0]
    d_out = dec_W.shape[1]

    def body(f_ref, cb_ref, a_ref, p_ref, w_ref, b_ref, g_ref, bb_ref,
             m_ref, v_ref, cl_ref, z_ref, de_ref, qd_ref):
        f = f_ref[...]
        cb = cb_ref[...]
        d2 = (jnp.sum(f * f, 1, keepdims=True)
              + jnp.sum(cb * cb, 1)[None, :]
              - 2.0 * lax.dot_general(f, cb, (((1,), (1,)), ((), ())),
                                      preferred_element_type=jnp.float32))
        mn = jnp.min(d2, axis=1, keepdims=True)
        iota = lax.broadcasted_iota(jnp.int32, d2.shape, 1)
        cand = jnp.where(d2 <= mn, iota, ncb)
        idx = jnp.min(cand, axis=1, keepdims=True)
        onehot = (iota == idx).astype(jnp.float32)
        quant = jnp.dot(onehot, cb, preferred_element_type=jnp.float32)
        mu = ((1.0 - AT) * (a_ref[0] + a_ref[1])
              + AT * (p_ref[0] + p_ref[1]))[:, :64]
        z = jnp.concatenate([quant, mu], axis=1)
        z_ref[...] = z
        de = jnp.dot(z, w_ref[...], preferred_element_type=jnp.float32)
        de_ref[...] = _bn_elu(de + b_ref[...], g_ref[...], bb_ref[...],
                              m_ref[...], v_ref[...])
        cl = cl_ref[...]
        dq = (jnp.sum(z * z, 1, keepdims=True)
              + jnp.sum(cl * cl, 1)[None, :]
              - 2.0 * lax.dot_general(z, cl, (((1,), (1,)), ((), ())),
                                      preferred_element_type=jnp.float32))
        q0 = 1.0 / (1.0 + dq)
        qd_ref[...] = q0 / jnp.sum(q0, axis=1, keepdims=True)

    full = lambda shape: pl.BlockSpec(shape, lambda i: (0,) * len(shape))
    fh2 = feat.shape[1]
    dmu = o2d.shape[2]
    ncl = cluster.shape[0]
    return pl.pallas_call(
        body,
        grid=(n // blk,),
        in_specs=[pl.BlockSpec((blk, fh2), lambda i: (i, 0)),
                  full((ncb, fh2)),
                  pl.BlockSpec((NC, blk, dmu), lambda i: (0, i, 0)),
                  pl.BlockSpec((NC, blk, dmu), lambda i: (0, i, 0)),
                  full((fh2 + 64, d_out)), full((d_out,)), full((d_out,)),
                  full((d_out,)), full((d_out,)), full((d_out,)),
                  full((ncl, fh2 + 64))],
        out_specs=[pl.BlockSpec((blk, fh2 + 64), lambda i: (i, 0)),
                   pl.BlockSpec((blk, d_out), lambda i: (i, 0)),
                   pl.BlockSpec((blk, ncl), lambda i: (i, 0))],
        out_shape=[jax.ShapeDtypeStruct((n, fh2 + 64), jnp.float32),
                   jax.ShapeDtypeStruct((n, d_out), jnp.float32),
                   jax.ShapeDtypeStruct((n, ncl), jnp.float32)],
    )(feat, codebook, o2d, o2e, dec_W, dec_b, bg, bb, bm, bv, cluster)


def kernel(x, adj, adj_prue, training,
           enc_W1, enc_b1, bn1_g, bn1_b, bn1_m, bn1_v,
           enc_W2, enc_b2, bn2_g, bn2_b, bn2_m, bn2_v,
           gc1_Wq, gc1_Wk, gc1_Wv, gc1_Ws, gc1_bq, gc1_bk, gc1_bv, gc1_bs,
           ch_Wq, ch_Wk, ch_Wv, ch_Ws, ch_bq, ch_bk, ch_bv, ch_bs,
           gc2_Wq, gc2_Wk, gc2_Wv, gc2_Ws, gc2_bq, gc2_bk, gc2_bv, gc2_bs,
           codebook, dec_W, dec_b, bnd_g, bnd_b, bnd_m, bnd_v, cluster):
    srcA, dstA = adj[0], adj[1]
    srcP, dstP = adj_prue[0], adj_prue[1]

    Wc1 = jnp.concatenate([gc1_Wq, gc1_Wk, gc1_Wv, gc1_Ws], axis=1)
    bc1 = jnp.concatenate([gc1_bq, gc1_bk, gc1_bv, gc1_bs])
    Wc2 = jnp.concatenate([ch_Wq, ch_Wk, ch_Wv, ch_Ws], axis=1)
    bc2 = jnp.concatenate([ch_bq, ch_bk, ch_bv, ch_bs])
    _pw = lambda W: jnp.pad(W, ((0, 0), (0, 64)))
    _pb = lambda b: jnp.pad(b, (0, 64))
    Wc3 = jnp.concatenate([_pw(gc2_Wq), _pw(gc2_Wk), _pw(gc2_Wv),
                           _pw(gc2_Ws)], axis=1)
    bc3 = jnp.concatenate([_pb(gc2_bq), _pb(gc2_bk), _pb(gc2_bv),
                           _pb(gc2_bs)])

    feat = _encoder(x, enc_W1, enc_b1, bn1_g, bn1_b, bn1_m, bn1_v,
                    enc_W2, enc_b2, bn2_g, bn2_b, bn2_m, bn2_v)

    _s128 = 1.0 / math.sqrt(128.0)
    _s64 = 1.0 / math.sqrt(64.0)

    q1, kv1, sk1 = _qkvs_direct(feat, Wc1, bc1, 128)
    exA, denA, aggA = _sc_ab(q1, kv1, srcA, dstA, d=128, scale=_s128)
    exP, denP, aggP = _sc_ab(q1, kv1, srcP, dstP, d=128, scale=_s128)
    h1, g1 = _comb(denA, aggA, sk1)
    h1p, g1p = _comb(denP, aggP, sk1)
    o2A = _sc_c(g1, srcA, dstA, exA, d=128)
    o2P = _sc_c(g1p, srcP, dstP, exP, d=128)

    q2, kv2, sk2 = _qkvs_mix(o2A, o2P, Wc2, bc2, 128)
    exB, denB, aggB = _sc_ab(q2, kv2, srcA, dstA, d=128, scale=_s128)
    x1, gB = _comb(denB, aggB, sk2)
    o2B = _sc_c(gB, srcA, dstA, exB, d=128)
    q3, kv3, sk3 = _qkvs_direct(x1, Wc2, bc2, 128)
    exC, denC, aggC = _sc_ab(q3, kv3, srcP, dstP, d=128, scale=_s128)
    xp, gC = _comb(denC, aggC, sk3)
    o2C = _sc_c(gC, srcP, dstP, exC, d=128)

    q4, kv4, sk4 = _qkvs_mix(o2B, o2C, Wc3, bc3, 128)
    exD, denD, aggD = _sc_ab(q4, kv4, srcA, dstA, d=128, scale=_s64)
    exE, denE, aggE = _sc_ab(q4, kv4, srcP, dstP, d=128, scale=_s64)
    muA, gD = _comb(denD, aggD, sk4)
    muP, gE = _comb(denE, aggE, sk4)
    o2D = _sc_c(gD, srcA, dstA, exD, d=128)
    o2E = _sc_c(gE, srcP, dstP, exE, d=128)

    z, de, qd = _decoder(feat, codebook, o2D, o2E, dec_W, dec_b,
                         bnd_g, bnd_b, bnd_m, bnd_v, cluster)
    return z, de, qd, feat


# speedup vs baseline: 10.4007x; 1.9439x over previous
import functools
import math

import jax
import jax.numpy as jnp
from jax import lax
from jax.experimental import pallas as pl
from jax.experimental.pallas import tpu as pltpu
from jax.experimental.pallas import tpu_sc as plsc

N = 4096
E = 65536
AT = 0.5
NC = 2
NS = 16
NW = NC * NS
L = 16
EPW = E // NW
CHK = 64
NCHUNK = EPW // CHK
ROWS = N // NS

_MESH = dict(core_axis_name="c", subcore_axis_name="s", num_cores=NC,
             num_subcores=NS)


def _zero_vmem_2d(ref, rows, d):
    zeros = jnp.zeros((L,), jnp.float32)

    def zrow(r, carry):
        for j in range(d // L):
            ref[r, pl.ds(j * L, L)] = zeros
        return carry

    lax.fori_loop(0, rows, zrow, 0)


def _zero_vmem_1d(ref, n):
    zeros = jnp.zeros((L,), jnp.float32)

    def zblk(i, carry):
        ref[pl.ds(i * L, L)] = zeros
        return carry

    lax.fori_loop(0, n // L, zblk, 0)


@functools.partial(jax.jit, static_argnames=("d", "scale"))
def _sc_ab(q, kv, src, dst, *, d, scale):
    def body(q_hbm, kv_hbm, src_hbm, dst_hbm, ex_hbm, den_hbm, agg_hbm,
             src_v0, src_v1, dst_v0, dst_v1, dst_s0, dst_s1,
             qrows0, qrows1, kvrows0, kvrows1, srows0, srows1, exb, tbuf,
             den_acc, agg_acc,
             sem_i0, sem_i1, sem_q0, sem_q1, sem_k0, sem_k1,
             sem_d0, sem_d1, sem_a0, sem_a1):
        c = lax.axis_index("c")
        s = lax.axis_index("s")
        wid = c * NS + s
        base = wid * EPW
        eidx = jnp.arange(L, dtype=jnp.int32)

        SRC = (src_v0, src_v1)
        DST = (dst_v0, dst_v1)
        DSC = (dst_s0, dst_s1)
        QR = (qrows0, qrows1)
        KVR = (kvrows0, kvrows1)
        SR = (srows0, srows1)
        SI = (sem_i0, sem_i1)
        SQ = (sem_q0, sem_q1)
        SK = (sem_k0, sem_k1)
        SD = (sem_d0, sem_d1)
        SA = (sem_a0, sem_a1)

        _zero_vmem_2d(srows0, CHK, d)
        _zero_vmem_1d(exb, ROWS)
        for t in range(ROWS // CHK):
            pltpu.sync_copy(srows0, agg_acc.at[pl.ds(s * ROWS + t * CHK, CHK)])
        pltpu.sync_copy(exb.at[pl.ds(0, ROWS)],
                        den_acc.at[pl.ds(s * ROWS, ROWS)])
        plsc.subcore_barrier()

        def idx_issue(i, r):
            st = base + i * CHK
            pltpu.async_copy(src_hbm.at[pl.ds(st, CHK)], SRC[r], SI[r])
            pltpu.async_copy(dst_hbm.at[pl.ds(st, CHK)], DST[r], SI[r])

        def idx_wait(i, r):
            st = base + i * CHK
            pltpu.make_async_copy(src_hbm.at[pl.ds(st, CHK)], SRC[r], SI[r]).wait()
            pltpu.make_async_copy(dst_hbm.at[pl.ds(st, CHK)], DST[r], SI[r]).wait()

        def gat_issue(r):
            pltpu.async_copy(q_hbm.at[DST[r]], QR[r], SQ[r])
            pltpu.async_copy(kv_hbm.at[SRC[r]], KVR[r], SK[r])

        def gat_wait(r):
            pltpu.make_async_copy(q_hbm.at[DST[r]], QR[r], SQ[r]).wait()
            pltpu.make_async_copy(kv_hbm.at[SRC[r]], KVR[r], SK[r]).wait()

        def sca_issue(i, r):
            pltpu.async_copy(exb.at[pl.ds(i * CHK, CHK)],
                             den_acc.at[DSC[r]], SD[r], add=True)
            pltpu.async_copy(SR[r], agg_acc.at[DSC[r]], SA[r], add=True)

        def sca_wait(i, r):
            pltpu.make_async_copy(exb.at[pl.ds(i * CHK, CHK)],
                                  den_acc.at[DSC[r]], SD[r]).wait()
            pltpu.make_async_copy(SR[r], agg_acc.at[DSC[r]], SA[r]).wait()

        def compute(i, r):
            qr, kvr = QR[r], KVR[r]
            fvs = [jnp.full((L,), j, jnp.int32) for j in range(L)]

            def cgrp(g, carry):
                row0 = g * L
                for el in range(L):
                    row = row0 + el
                    acc = qr[row, pl.ds(0, L)] * kvr[row, pl.ds(0, L)]
                    for j in range(1, d // L):
                        acc = acc + (qr[row, pl.ds(j * L, L)]
                                     * kvr[row, pl.ds(j * L, L)])
                    tbuf[el, :] = acc
                tot = plsc.load_gather(tbuf, [eidx, fvs[0]])
                for j in range(1, L):
                    tot = tot + plsc.load_gather(tbuf, [eidx, fvs[j]])
                exb[pl.ds(i * CHK + g * L, L)] = jnp.exp(tot * scale)
                return carry

            lax.fori_loop(0, CHK // L, cgrp, 0)

        def snap_idx(r):
            for t in range(CHK // L):
                DSC[r][pl.ds(t * L, L)] = DST[r][pl.ds(t * L, L)]

        def scale_rows(i, r):
            kvr, sr = KVR[r], SR[r]

            def sgrp(g2, carry2):
                exv = exb[pl.ds(i * CHK + g2 * L, L)]
                for el in range(L):
                    sc = exv[el]
                    for j in range(d // L):
                        sr[g2 * L + el, pl.ds(j * L, L)] = (
                            kvr[g2 * L + el, pl.ds(d + j * L, L)] * sc)
                return carry2

            lax.fori_loop(0, CHK // L, sgrp, 0)

        def one(i, r, rn):
            @pl.when(i + 1 < NCHUNK)
            def _():
                idx_wait(i + 1, rn)
                gat_issue(rn)
            gat_wait(r)
            snap_idx(r)
            @pl.when(i + 2 < NCHUNK)
            def _():
                idx_issue(i + 2, r)
            compute(i, r)
            @pl.when(i >= 2)
            def _():
                sca_wait(i - 2, r)
            scale_rows(i, r)
            sca_issue(i, r)

        idx_issue(0, 0)
        idx_wait(0, 0)
        gat_issue(0)
        idx_issue(1, 1)

        def pair(j, carry):
            one(2 * j, 0, 1)
            one(2 * j + 1, 1, 0)
            return carry

        lax.fori_loop(0, NCHUNK // 2, pair, 0)
        sca_wait(NCHUNK - 2, 0)
        sca_wait(NCHUNK - 1, 1)
        pltpu.sync_copy(exb, ex_hbm.at[pl.ds(base, EPW)])
        plsc.subcore_barrier()
        pltpu.sync_copy(den_acc.at[pl.ds(s * ROWS, ROWS)],
                        den_hbm.at[c, pl.ds(s * ROWS, ROWS)])
        pltpu.sync_copy(agg_acc.at[pl.ds(s * ROWS, ROWS)],
                        agg_hbm.at[c, pl.ds(s * ROWS, ROWS)])

    f = pl.kernel(
        body,
        out_type=[
            jax.ShapeDtypeStruct((E,), jnp.float32),
            jax.ShapeDtypeStruct((NC, N), jnp.float32),
            jax.ShapeDtypeStruct((NC, N, d), jnp.float32),
        ],
        mesh=plsc.VectorSubcoreMesh(**_MESH),
        compiler_params=pltpu.CompilerParams(needs_layout_passes=False),
        scratch_types=(
            [pltpu.VMEM((CHK,), jnp.int32)] * 6
            + [pltpu.VMEM((CHK, d), jnp.float32)] * 2
            + [pltpu.VMEM((CHK, 2 * d), jnp.float32)] * 2
            + [pltpu.VMEM((CHK, d), jnp.float32)] * 2
            + [pltpu.VMEM((EPW,), jnp.float32)]
            + [pltpu.VMEM((L, L), jnp.float32)]
            + [pltpu.VMEM_SHARED((N,), jnp.float32),
               pltpu.VMEM_SHARED((N, d), jnp.float32)]
            + [pltpu.SemaphoreType.DMA] * 10
        ),
    )
    return f(q, kv, src, dst)


@functools.partial(jax.jit, static_argnames=("d",))
def _sc_c(g, src, dst, ex, *, d):
    def body(g_hbm, src_hbm, dst_hbm, ex_hbm, out_hbm,
             src_v0, src_v1, dst_v0, dst_v1, src_s0, src_s1,
             grows0, grows1, srows0, srows1, exb0, exb1, exk,
             acc,
             sem_i0, sem_i1, sem_g0, sem_g1, sem_a0, sem_a1):
        c = lax.axis_index("c")
        s = lax.axis_index("s")
        wid = c * NS + s
        base = wid * EPW

        SRC = (src_v0, src_v1)
        DST = (dst_v0, dst_v1)
        SSC = (src_s0, src_s1)
        GR = (grows0, grows1)
        SR = (srows0, srows1)
        EXB = (exb0, exb1)
        SI = (sem_i0, sem_i1)
        SG = (sem_g0, sem_g1)
        SA = (sem_a0, sem_a1)

        _zero_vmem_2d(srows0, CHK, d)
        for t in range(ROWS // CHK):
            pltpu.sync_copy(srows0, acc.at[pl.ds(s * ROWS + t * CHK, CHK)])
        plsc.subcore_barrier()

        def idx_issue(i, r):
            st = base + i * CHK
            pltpu.async_copy(src_hbm.at[pl.ds(st, CHK)], SRC[r], SI[r])
            pltpu.async_copy(dst_hbm.at[pl.ds(st, CHK)], DST[r], SI[r])
            pltpu.async_copy(ex_hbm.at[pl.ds(st, CHK)], EXB[r], SI[r])

        def idx_wait(i, r):
            st = base + i * CHK
            pltpu.make_async_copy(src_hbm.at[pl.ds(st, CHK)], SRC[r], SI[r]).wait()
            pltpu.make_async_copy(dst_hbm.at[pl.ds(st, CHK)], DST[r], SI[r]).wait()
            pltpu.make_async_copy(ex_hbm.at[pl.ds(st, CHK)], EXB[r], SI[r]).wait()

        def gat_issue(r):
            pltpu.async_copy(g_hbm.at[DST[r]], GR[r], SG[r])

        def gat_wait(r):
            pltpu.make_async_copy(g_hbm.at[DST[r]], GR[r], SG[r]).wait()

        def sca_issue(r):
            pltpu.async_copy(SR[r], acc.at[SSC[r]], SA[r], add=True)

        def sca_wait(r):
            pltpu.make_async_copy(SR[r], acc.at[SSC[r]], SA[r]).wait()

        def snap_idx(r):
            for t in range(CHK // L):
                sl = pl.ds(t * L, L)
                SSC[r][sl] = SRC[r][sl]
                keep = SRC[r][sl] != DST[r][sl]
                exk[sl] = jnp.where(keep, EXB[r][sl], 0.0)

        def scale_rows(r):
            def sgrp(g2, carry2):
                exv = exk[pl.ds(g2 * L, L)]
                for el in range(L):
                    sc = exv[el]
                    for j in range(d // L):
                        SR[r][g2 * L + el, pl.ds(j * L, L)] = (
                            GR[r][g2 * L + el, pl.ds(j * L, L)] * sc)
                return carry2

            lax.fori_loop(0, CHK // L, sgrp, 0)

        def one(i, r, rn):
            @pl.when(i + 1 < NCHUNK)
            def _():
                idx_wait(i + 1, rn)
                gat_issue(rn)
            gat_wait(r)
            snap_idx(r)
            @pl.when(i + 2 < NCHUNK)
            def _():
                idx_issue(i + 2, r)
            @pl.when(i >= 2)
            def _():
                sca_wait(r)
            scale_rows(r)
            sca_issue(r)

        idx_issue(0, 0)
        idx_wait(0, 0)
        gat_issue(0)
        idx_issue(1, 1)

        def pair(j, carry):
            one(2 * j, 0, 1)
            one(2 * j + 1, 1, 0)
            return carry

        lax.fori_loop(0, NCHUNK // 2, pair, 0)
        sca_wait(0)
        sca_wait(1)
        plsc.subcore_barrier()
        pltpu.sync_copy(acc.at[pl.ds(s * ROWS, ROWS)],
                        out_hbm.at[c, pl.ds(s * ROWS, ROWS)])

    f = pl.kernel(
        body,
        out_type=[jax.ShapeDtypeStruct((NC, N, d), jnp.float32)],
        mesh=plsc.VectorSubcoreMesh(**_MESH),
        compiler_params=pltpu.CompilerParams(needs_layout_passes=False),
        scratch_types=(
            [pltpu.VMEM((CHK,), jnp.int32)] * 6
            + [pltpu.VMEM((CHK, d), jnp.float32)] * 4
            + [pltpu.VMEM((CHK,), jnp.float32)] * 3
            + [pltpu.VMEM_SHARED((N, d), jnp.float32)]
            + [pltpu.SemaphoreType.DMA] * 6
        ),
    )
    return f(g, src, dst, ex)[0]


def _bn_elu(y, g, b, m, v):
    y = g * (y - m) * lax.rsqrt(v + 1e-4) + b
    return jnp.where(y > 0, y, jnp.exp(y) - 1.0)


def _enc_body(x_ref, W1_ref, b1_ref, g1_ref, bb1_ref, m1_ref, v1_ref,
              W2_ref, b2_ref, g2_ref, bb2_ref, m2_ref, v2_ref, out_ref):
    y = jnp.dot(x_ref[...], W1_ref[...], preferred_element_type=jnp.float32)
    y = _bn_elu(y + b1_ref[...], g1_ref[...], bb1_ref[...], m1_ref[...],
                v1_ref[...])
    z = jnp.dot(y, W2_ref[...], preferred_element_type=jnp.float32)
    z = _bn_elu(z + b2_ref[...], g2_ref[...], bb2_ref[...], m2_ref[...],
                v2_ref[...])
    out_ref[...] = z


def _encoder(x, W1, b1, g1, bb1, m1, v1, W2, b2, g2, bb2, m2, v2):
    n, d_in = x.shape
    fh1 = W1.shape[1]
    fh2 = W2.shape[1]
    blk = 512
    full = lambda shape: pl.BlockSpec(shape, lambda i: (0,) * len(shape))
    return pl.pallas_call(
        _enc_body,
        grid=(n // blk,),
        in_specs=[
            pl.BlockSpec((blk, d_in), lambda i: (i, 0)),
            full((d_in, fh1)), full((fh1,)), full((fh1,)), full((fh1,)),
            full((fh1,)), full((fh1,)),
            full((fh1, fh2)), full((fh2,)), full((fh2,)), full((fh2,)),
            full((fh2,)), full((fh2,)),
        ],
        out_specs=pl.BlockSpec((blk, fh2), lambda i: (i, 0)),
        out_shape=jax.ShapeDtypeStruct((n, fh2), jnp.float32),
    )(x, W1, b1, g1, bb1, m1, v1, W2, b2, g2, bb2, m2, v2)


def _qkvs_direct(x, Wcat, bcat, d):
    n, din = x.shape
    blk = 1024

    def body(x_ref, w_ref, b_ref, q_ref, kv_ref, s_ref):
        y = jnp.dot(x_ref[...], w_ref[...],
                    preferred_element_type=jnp.float32) + b_ref[...]
        q_ref[...] = y[:, :d]
        kv_ref[...] = y[:, d:3 * d]
        s_ref[...] = y[:, 3 * d:]

    full = lambda shape: pl.BlockSpec(shape, lambda i: (0,) * len(shape))
    return pl.pallas_call(
        body,
        grid=(n // blk,),
        in_specs=[pl.BlockSpec((blk, din), lambda i: (i, 0)),
                  full((din, 4 * d)), full((4 * d,))],
        out_specs=[pl.BlockSpec((blk, d), lambda i: (i, 0)),
                   pl.BlockSpec((blk, 2 * d), lambda i: (i, 0)),
                   pl.BlockSpec((blk, d), lambda i: (i, 0))],
        out_shape=[jax.ShapeDtypeStruct((n, d), jnp.float32),
                   jax.ShapeDtypeStruct((n, 2 * d), jnp.float32),
                   jax.ShapeDtypeStruct((n, d), jnp.float32)],
    )(x, Wcat, bcat)


def _qkvs_mix(o2a, o2p, Wcat, bcat, d):
    _, n, din = o2a.shape
    blk = 1024

    def body(a_ref, p_ref, w_ref, b_ref, q_ref, kv_ref, s_ref):
        xin = ((1.0 - AT) * (a_ref[0] + a_ref[1])
               + AT * (p_ref[0] + p_ref[1]))
        xin = jnp.maximum(xin, 0.0)
        y = jnp.dot(xin, w_ref[...],
                    preferred_element_type=jnp.float32) + b_ref[...]
        q_ref[...] = y[:, :d]
        kv_ref[...] = y[:, d:3 * d]
        s_ref[...] = y[:, 3 * d:]

    full = lambda shape: pl.BlockSpec(shape, lambda i: (0,) * len(shape))
    return pl.pallas_call(
        body,
        grid=(n // blk,),
        in_specs=[pl.BlockSpec((NC, blk, din), lambda i: (0, i, 0)),
                  pl.BlockSpec((NC, blk, din), lambda i: (0, i, 0)),
                  full((din, 4 * d)), full((4 * d,))],
        out_specs=[pl.BlockSpec((blk, d), lambda i: (i, 0)),
                   pl.BlockSpec((blk, 2 * d), lambda i: (i, 0)),
                   pl.BlockSpec((blk, d), lambda i: (i, 0))],
        out_shape=[jax.ShapeDtypeStruct((n, d), jnp.float32),
                   jax.ShapeDtypeStruct((n, 2 * d), jnp.float32),
                   jax.ShapeDtypeStruct((n, d), jnp.float32)],
    )(o2a, o2p, Wcat, bcat)


def _comb(den, agg, skip):
    _, n, d = agg.shape
    blk = 1024

    def body(den_ref, agg_ref, skip_ref, h_ref, g_ref):
        inv = 1.0 / (den_ref[0, :] + den_ref[1, :] + 1e-16)
        h = (agg_ref[0] + agg_ref[1]) * inv[:, None] + skip_ref[...]
        h_ref[...] = h
        g_ref[...] = h * inv[:, None]

    return pl.pallas_call(
        body,
        grid=(n // blk,),
        in_specs=[pl.BlockSpec((NC, blk), lambda i: (0, i)),
                  pl.BlockSpec((NC, blk, d), lambda i: (0, i, 0)),
                  pl.BlockSpec((blk, d), lambda i: (i, 0))],
        out_specs=[pl.BlockSpec((blk, d), lambda i: (i, 0)),
                   pl.BlockSpec((blk, d), lambda i: (i, 0))],
        out_shape=[jax.ShapeDtypeStruct((n, d), jnp.float32),
                   jax.ShapeDtypeStruct((n, d), jnp.float32)],
    )(den, agg, skip)


def _decoder(feat, codebook, o2d, o2e, dec_W, dec_b, bg, bb, bm, bv, cluster):
    n = feat.shape[0]
    blk = 512
    ncb = codebook.shape[0]
    d_out = dec_W.shape[1]

    def body(f_ref, cb_ref, a_ref, p_ref, w_ref, b_ref, g_ref, bb_ref,
             m_ref, v_ref, cl_ref, z_ref, de_ref, qd_ref):
        f = f_ref[...]
        cb = cb_ref[...]
        d2 = (jnp.sum(f * f, 1, keepdims=True)
              + jnp.sum(cb * cb, 1)[None, :]
              - 2.0 * lax.dot_general(f, cb, (((1,), (1,)), ((), ())),
                                      preferred_element_type=jnp.float32))
        mn = jnp.min(d2, axis=1, keepdims=True)
        iota = lax.broadcasted_iota(jnp.int32, d2.shape, 1)
        cand = jnp.where(d2 <= mn, iota, ncb)
        idx = jnp.min(cand, axis=1, keepdims=True)
        onehot = (iota == idx).astype(jnp.float32)
        quant = jnp.dot(onehot, cb, preferred_element_type=jnp.float32)
        mu = ((1.0 - AT) * (a_ref[0] + a_ref[1])
              + AT * (p_ref[0] + p_ref[1]))[:, :64]
        z = jnp.concatenate([quant, mu], axis=1)
        z_ref[...] = z
        de = jnp.dot(z, w_ref[...], preferred_element_type=jnp.float32)
        de_ref[...] = _bn_elu(de + b_ref[...], g_ref[...], bb_ref[...],
                              m_ref[...], v_ref[...])
        cl = cl_ref[...]
        dq = (jnp.sum(z * z, 1, keepdims=True)
              + jnp.sum(cl * cl, 1)[None, :]
              - 2.0 * lax.dot_general(z, cl, (((1,), (1,)), ((), ())),
                                      preferred_element_type=jnp.float32))
        q0 = 1.0 / (1.0 + dq)
        qd_ref[...] = q0 / jnp.sum(q0, axis=1, keepdims=True)

    full = lambda shape: pl.BlockSpec(shape, lambda i: (0,) * len(shape))
    fh2 = feat.shape[1]
    dmu = o2d.shape[2]
    ncl = cluster.shape[0]
    return pl.pallas_call(
        body,
        grid=(n // blk,),
        in_specs=[pl.BlockSpec((blk, fh2), lambda i: (i, 0)),
                  full((ncb, fh2)),
                  pl.BlockSpec((NC, blk, dmu), lambda i: (0, i, 0)),
                  pl.BlockSpec((NC, blk, dmu), lambda i: (0, i, 0)),
                  full((fh2 + 64, d_out)), full((d_out,)), full((d_out,)),
                  full((d_out,)), full((d_out,)), full((d_out,)),
                  full((ncl, fh2 + 64))],
        out_specs=[pl.BlockSpec((blk, fh2 + 64), lambda i: (i, 0)),
                   pl.BlockSpec((blk, d_out), lambda i: (i, 0)),
                   pl.BlockSpec((blk, ncl), lambda i: (i, 0))],
        out_shape=[jax.ShapeDtypeStruct((n, fh2 + 64), jnp.float32),
                   jax.ShapeDtypeStruct((n, d_out), jnp.float32),
                   jax.ShapeDtypeStruct((n, ncl), jnp.float32)],
    )(feat, codebook, o2d, o2e, dec_W, dec_b, bg, bb, bm, bv, cluster)


def kernel(x, adj, adj_prue, training,
           enc_W1, enc_b1, bn1_g, bn1_b, bn1_m, bn1_v,
           enc_W2, enc_b2, bn2_g, bn2_b, bn2_m, bn2_v,
           gc1_Wq, gc1_Wk, gc1_Wv, gc1_Ws, gc1_bq, gc1_bk, gc1_bv, gc1_bs,
           ch_Wq, ch_Wk, ch_Wv, ch_Ws, ch_bq, ch_bk, ch_bv, ch_bs,
           gc2_Wq, gc2_Wk, gc2_Wv, gc2_Ws, gc2_bq, gc2_bk, gc2_bv, gc2_bs,
           codebook, dec_W, dec_b, bnd_g, bnd_b, bnd_m, bnd_v, cluster):
    srcA, dstA = adj[0], adj[1]
    srcP, dstP = adj_prue[0], adj_prue[1]

    Wc1 = jnp.concatenate([gc1_Wq, gc1_Wk, gc1_Wv, gc1_Ws], axis=1)
    bc1 = jnp.concatenate([gc1_bq, gc1_bk, gc1_bv, gc1_bs])
    Wc2 = jnp.concatenate([ch_Wq, ch_Wk, ch_Wv, ch_Ws], axis=1)
    bc2 = jnp.concatenate([ch_bq, ch_bk, ch_bv, ch_bs])
    _pw = lambda W: jnp.pad(W, ((0, 0), (0, 64)))
    _pb = lambda b: jnp.pad(b, (0, 64))
    Wc3 = jnp.concatenate([_pw(gc2_Wq), _pw(gc2_Wk), _pw(gc2_Wv),
                           _pw(gc2_Ws)], axis=1)
    bc3 = jnp.concatenate([_pb(gc2_bq), _pb(gc2_bk), _pb(gc2_bv),
                           _pb(gc2_bs)])

    feat = _encoder(x, enc_W1, enc_b1, bn1_g, bn1_b, bn1_m, bn1_v,
                    enc_W2, enc_b2, bn2_g, bn2_b, bn2_m, bn2_v)

    _s128 = 1.0 / math.sqrt(128.0)
    _s64 = 1.0 / math.sqrt(64.0)

    q1, kv1, sk1 = _qkvs_direct(feat, Wc1, bc1, 128)
    exA, denA, aggA = _sc_ab(q1, kv1, srcA, dstA, d=128, scale=_s128)
    exP, denP, aggP = _sc_ab(q1, kv1, srcP, dstP, d=128, scale=_s128)
    h1, g1 = _comb(denA, aggA, sk1)
    h1p, g1p = _comb(denP, aggP, sk1)
    o2A = _sc_c(g1, srcA, dstA, exA, d=128)
    o2P = _sc_c(g1p, srcP, dstP, exP, d=128)

    q2, kv2, sk2 = _qkvs_mix(o2A, o2P, Wc2, bc2, 128)
    exB, denB, aggB = _sc_ab(q2, kv2, srcA, dstA, d=128, scale=_s128)
    x1, gB = _comb(denB, aggB, sk2)
    o2B = _sc_c(gB, srcA, dstA, exB, d=128)
    q3, kv3, sk3 = _qkvs_direct(x1, Wc2, bc2, 128)
    exC, denC, aggC = _sc_ab(q3, kv3, srcP, dstP, d=128, scale=_s128)
    xp, gC = _comb(denC, aggC, sk3)
    o2C = _sc_c(gC, srcP, dstP, exC, d=128)

    q4, kv4, sk4 = _qkvs_mix(o2B, o2C, Wc3, bc3, 128)
    exD, denD, aggD = _sc_ab(q4, kv4, srcA, dstA, d=128, scale=_s64)
    exE, denE, aggE = _sc_ab(q4, kv4, srcP, dstP, d=128, scale=_s64)
    muA, gD = _comb(denD, aggD, sk4)
    muP, gE = _comb(denE, aggE, sk4)
    o2D = _sc_c(gD, srcA, dstA, exD, d=128)
    o2E = _sc_c(gE, srcP, dstP, exE, d=128)

    z, de, qd = _decoder(feat, codebook, o2D, o2E, dec_W, dec_b,
                         bnd_g, bnd_b, bnd_m, bnd_v, cluster)
    return z, de, qd, feat


# speedup vs baseline: 13.5403x; 1.3019x over previous
import functools
import math

import jax
import jax.numpy as jnp
from jax import lax
from jax.experimental import pallas as pl
from jax.experimental.pallas import tpu as pltpu
from jax.experimental.pallas import tpu_sc as plsc

N = 4096
E = 65536
AT = 0.5
NC = 2
NS = 16
NW = NC * NS
L = 16
EPW = E // NW
CHK = 64
NCHUNK = EPW // CHK
ROWS = N // NS

_MESH = dict(core_axis_name="c", subcore_axis_name="s", num_cores=NC,
             num_subcores=NS)


def _zero_vmem_2d(ref, rows, d):
    zeros = jnp.zeros((L,), jnp.float32)

    def zrow(r, carry):
        for j in range(d // L):
            ref[r, pl.ds(j * L, L)] = zeros
        return carry

    lax.fori_loop(0, rows, zrow, 0)


def _zero_vmem_1d(ref, n):
    zeros = jnp.zeros((L,), jnp.float32)

    def zblk(i, carry):
        ref[pl.ds(i * L, L)] = zeros
        return carry

    lax.fori_loop(0, n // L, zblk, 0)


@functools.partial(jax.jit, static_argnames=("d",))
def _sc_ab(exf, v, src, dst, *, d):
    def body(exf_hbm, v_hbm, src_hbm, dst_hbm, ex_hbm, den_hbm, agg_hbm,
             src_v0, src_v1, dst_v0, dst_v1, dst_s0, dst_s1, exi0, exi1,
             vrows0, vrows1, srows0, srows1, exb,
             den_acc, agg_acc,
             sem_i0, sem_i1, sem_e0, sem_e1, sem_v0, sem_v1,
             sem_d0, sem_d1, sem_a0, sem_a1):
        c = lax.axis_index("c")
        s = lax.axis_index("s")
        wid = c * NS + s
        base = wid * EPW

        SRC = (src_v0, src_v1)
        DST = (dst_v0, dst_v1)
        DSC = (dst_s0, dst_s1)
        EXI = (exi0, exi1)
        VR = (vrows0, vrows1)
        SR = (srows0, srows1)
        SI = (sem_i0, sem_i1)
        SE = (sem_e0, sem_e1)
        SV = (sem_v0, sem_v1)
        SD = (sem_d0, sem_d1)
        SA = (sem_a0, sem_a1)

        _zero_vmem_2d(srows0, CHK, d)
        _zero_vmem_1d(exb, ROWS)
        for t in range(ROWS // CHK):
            pltpu.sync_copy(srows0, agg_acc.at[pl.ds(s * ROWS + t * CHK, CHK)])
        pltpu.sync_copy(exb.at[pl.ds(0, ROWS)],
                        den_acc.at[pl.ds(s * ROWS, ROWS)])
        plsc.subcore_barrier()

        def idx_issue(i, r):
            st = base + i * CHK
            pltpu.async_copy(src_hbm.at[pl.ds(st, CHK)], SRC[r], SI[r])
            pltpu.async_copy(dst_hbm.at[pl.ds(st, CHK)], DST[r], SI[r])

        def idx_wait(i, r):
            st = base + i * CHK
            pltpu.make_async_copy(src_hbm.at[pl.ds(st, CHK)], SRC[r], SI[r]).wait()
            pltpu.make_async_copy(dst_hbm.at[pl.ds(st, CHK)], DST[r], SI[r]).wait()

        def exi_compute(r):
            for t in range(CHK // L):
                sl = pl.ds(t * L, L)
                EXI[r][sl] = DST[r][sl] * N + SRC[r][sl]

        def gat_issue(i, r):
            pltpu.async_copy(exf_hbm.at[EXI[r]], exb.at[pl.ds(i * CHK, CHK)],
                             SE[r])
            pltpu.async_copy(v_hbm.at[SRC[r]], VR[r], SV[r])

        def gat_wait(i, r):
            pltpu.make_async_copy(exf_hbm.at[EXI[r]],
                                  exb.at[pl.ds(i * CHK, CHK)], SE[r]).wait()
            pltpu.make_async_copy(v_hbm.at[SRC[r]], VR[r], SV[r]).wait()

        def sca_issue(i, r):
            pltpu.async_copy(exb.at[pl.ds(i * CHK, CHK)],
                             den_acc.at[DSC[r]], SD[r], add=True)
            pltpu.async_copy(SR[r], agg_acc.at[DSC[r]], SA[r], add=True)

        def sca_wait(i, r):
            pltpu.make_async_copy(exb.at[pl.ds(i * CHK, CHK)],
                                  den_acc.at[DSC[r]], SD[r]).wait()
            pltpu.make_async_copy(SR[r], agg_acc.at[DSC[r]], SA[r]).wait()

        def snap_idx(r):
            for t in range(CHK // L):
                DSC[r][pl.ds(t * L, L)] = DST[r][pl.ds(t * L, L)]

        def scale_rows(i, r):
            vr, sr = VR[r], SR[r]

            def sgrp(g2, carry2):
                exv = exb[pl.ds(i * CHK + g2 * L, L)]
                for el in range(L):
                    sc = exv[el]
                    for j in range(d // L):
                        sr[g2 * L + el, pl.ds(j * L, L)] = (
                            vr[g2 * L + el, pl.ds(j * L, L)] * sc)
                return carry2

            lax.fori_loop(0, CHK // L, sgrp, 0)

        def one(i, r, rn):
            @pl.when(i + 1 < NCHUNK)
            def _():
                idx_wait(i + 1, rn)
                exi_compute(rn)
                gat_issue(i + 1, rn)
            gat_wait(i, r)
            snap_idx(r)
            @pl.when(i + 2 < NCHUNK)
            def _():
                idx_issue(i + 2, r)
            @pl.when(i >= 2)
            def _():
                sca_wait(i - 2, r)
            scale_rows(i, r)
            sca_issue(i, r)

        idx_issue(0, 0)
        idx_wait(0, 0)
        exi_compute(0)
        gat_issue(0, 0)
        idx_issue(1, 1)

        def pair(j, carry):
            one(2 * j, 0, 1)
            one(2 * j + 1, 1, 0)
            return carry

        lax.fori_loop(0, NCHUNK // 2, pair, 0)
        sca_wait(NCHUNK - 2, 0)
        sca_wait(NCHUNK - 1, 1)
        pltpu.sync_copy(exb, ex_hbm.at[pl.ds(base, EPW)])
        plsc.subcore_barrier()
        pltpu.sync_copy(den_acc.at[pl.ds(s * ROWS, ROWS)],
                        den_hbm.at[c, pl.ds(s * ROWS, ROWS)])
        pltpu.sync_copy(agg_acc.at[pl.ds(s * ROWS, ROWS)],
                        agg_hbm.at[c, pl.ds(s * ROWS, ROWS)])

    f = pl.kernel(
        body,
        out_type=[
            jax.ShapeDtypeStruct((E,), jnp.float32),
            jax.ShapeDtypeStruct((NC, N), jnp.float32),
            jax.ShapeDtypeStruct((NC, N, d), jnp.float32),
        ],
        mesh=plsc.VectorSubcoreMesh(**_MESH),
        compiler_params=pltpu.CompilerParams(needs_layout_passes=False),
        scratch_types=(
            [pltpu.VMEM((CHK,), jnp.int32)] * 8
            + [pltpu.VMEM((CHK, d), jnp.float32)] * 4
            + [pltpu.VMEM((EPW,), jnp.float32)]
            + [pltpu.VMEM_SHARED((N,), jnp.float32),
               pltpu.VMEM_SHARED((N, d), jnp.float32)]
            + [pltpu.SemaphoreType.DMA] * 10
        ),
    )
    return f(exf, v, src, dst)


def _ex_mat(q, k, scale):
    n, d = q.shape
    blk = 512

    def body(q_ref, k_ref, out_ref):
        y = lax.dot_general(q_ref[...], k_ref[...], (((1,), (1,)), ((), ())),
                            preferred_element_type=jnp.float32)
        out_ref[...] = jnp.exp(y * scale)

    return pl.pallas_call(
        body,
        grid=(n // blk,),
        in_specs=[pl.BlockSpec((blk, d), lambda i: (i, 0)),
                  pl.BlockSpec((n, d), lambda i: (0, 0))],
        out_specs=pl.BlockSpec((blk, n), lambda i: (i, 0)),
        out_shape=jax.ShapeDtypeStruct((n, n), jnp.float32),
    )(q, k)


@functools.partial(jax.jit, static_argnames=("d",))
def _sc_c(g, src, dst, ex, *, d):
    def body(g_hbm, src_hbm, dst_hbm, ex_hbm, out_hbm,
             src_v0, src_v1, dst_v0, dst_v1, src_s0, src_s1,
             grows0, grows1, srows0, srows1, exb0, exb1, exk,
             acc,
             sem_i0, sem_i1, sem_g0, sem_g1, sem_a0, sem_a1):
        c = lax.axis_index("c")
        s = lax.axis_index("s")
        wid = c * NS + s
        base = wid * EPW

        SRC = (src_v0, src_v1)
        DST = (dst_v0, dst_v1)
        SSC = (src_s0, src_s1)
        GR = (grows0, grows1)
        SR = (srows0, srows1)
        EXB = (exb0, exb1)
        SI = (sem_i0, sem_i1)
        SG = (sem_g0, sem_g1)
        SA = (sem_a0, sem_a1)

        _zero_vmem_2d(srows0, CHK, d)
        for t in range(ROWS // CHK):
            pltpu.sync_copy(srows0, acc.at[pl.ds(s * ROWS + t * CHK, CHK)])
        plsc.subcore_barrier()

        def idx_issue(i, r):
            st = base + i * CHK
            pltpu.async_copy(src_hbm.at[pl.ds(st, CHK)], SRC[r], SI[r])
            pltpu.async_copy(dst_hbm.at[pl.ds(st, CHK)], DST[r], SI[r])
            pltpu.async_copy(ex_hbm.at[pl.ds(st, CHK)], EXB[r], SI[r])

        def idx_wait(i, r):
            st = base + i * CHK
            pltpu.make_async_copy(src_hbm.at[pl.ds(st, CHK)], SRC[r], SI[r]).wait()
            pltpu.make_async_copy(dst_hbm.at[pl.ds(st, CHK)], DST[r], SI[r]).wait()
            pltpu.make_async_copy(ex_hbm.at[pl.ds(st, CHK)], EXB[r], SI[r]).wait()

        def gat_issue(r):
            pltpu.async_copy(g_hbm.at[DST[r]], GR[r], SG[r])

        def gat_wait(r):
            pltpu.make_async_copy(g_hbm.at[DST[r]], GR[r], SG[r]).wait()

        def sca_issue(r):
            pltpu.async_copy(SR[r], acc.at[SSC[r]], SA[r], add=True)

        def sca_wait(r):
            pltpu.make_async_copy(SR[r], acc.at[SSC[r]], SA[r]).wait()

        def snap_idx(r):
            for t in range(CHK // L):
                sl = pl.ds(t * L, L)
                SSC[r][sl] = SRC[r][sl]
                keep = SRC[r][sl] != DST[r][sl]
                exk[sl] = jnp.where(keep, EXB[r][sl], 0.0)

        def scale_rows(r):
            def sgrp(g2, carry2):
                exv = exk[pl.ds(g2 * L, L)]
                for el in range(L):
                    sc = exv[el]
                    for j in range(d // L):
                        SR[r][g2 * L + el, pl.ds(j * L, L)] = (
                            GR[r][g2 * L + el, pl.ds(j * L, L)] * sc)
                return carry2

            lax.fori_loop(0, CHK // L, sgrp, 0)

        def one(i, r, rn):
            @pl.when(i + 1 < NCHUNK)
            def _():
                idx_wait(i + 1, rn)
                gat_issue(rn)
            gat_wait(r)
            snap_idx(r)
            @pl.when(i + 2 < NCHUNK)
            def _():
                idx_issue(i + 2, r)
            @pl.when(i >= 2)
            def _():
                sca_wait(r)
            scale_rows(r)
            sca_issue(r)

        idx_issue(0, 0)
        idx_wait(0, 0)
        gat_issue(0)
        idx_issue(1, 1)

        def pair(j, carry):
            one(2 * j, 0, 1)
            one(2 * j + 1, 1, 0)
            return carry

        lax.fori_loop(0, NCHUNK // 2, pair, 0)
        sca_wait(0)
        sca_wait(1)
        plsc.subcore_barrier()
        pltpu.sync_copy(acc.at[pl.ds(s * ROWS, ROWS)],
                        out_hbm.at[c, pl.ds(s * ROWS, ROWS)])

    f = pl.kernel(
        body,
        out_type=[jax.ShapeDtypeStruct((NC, N, d), jnp.float32)],
        mesh=plsc.VectorSubcoreMesh(**_MESH),
        compiler_params=pltpu.CompilerParams(needs_layout_passes=False),
        scratch_types=(
            [pltpu.VMEM((CHK,), jnp.int32)] * 6
            + [pltpu.VMEM((CHK, d), jnp.float32)] * 4
            + [pltpu.VMEM((CHK,), jnp.float32)] * 3
            + [pltpu.VMEM_SHARED((N, d), jnp.float32)]
            + [pltpu.SemaphoreType.DMA] * 6
        ),
    )
    return f(g, src, dst, ex)[0]


def _bn_elu(y, g, b, m, v):
    y = g * (y - m) * lax.rsqrt(v + 1e-4) + b
    return jnp.where(y > 0, y, jnp.exp(y) - 1.0)


def _enc_body(x_ref, W1_ref, b1_ref, g1_ref, bb1_ref, m1_ref, v1_ref,
              W2_ref, b2_ref, g2_ref, bb2_ref, m2_ref, v2_ref, out_ref):
    y = jnp.dot(x_ref[...], W1_ref[...], preferred_element_type=jnp.float32)
    y = _bn_elu(y + b1_ref[...], g1_ref[...], bb1_ref[...], m1_ref[...],
                v1_ref[...])
    z = jnp.dot(y, W2_ref[...], preferred_element_type=jnp.float32)
    z = _bn_elu(z + b2_ref[...], g2_ref[...], bb2_ref[...], m2_ref[...],
                v2_ref[...])
    out_ref[...] = z


def _encoder(x, W1, b1, g1, bb1, m1, v1, W2, b2, g2, bb2, m2, v2):
    n, d_in = x.shape
    fh1 = W1.shape[1]
    fh2 = W2.shape[1]
    blk = 512
    full = lambda shape: pl.BlockSpec(shape, lambda i: (0,) * len(shape))
    return pl.pallas_call(
        _enc_body,
        grid=(n // blk,),
        in_specs=[
            pl.BlockSpec((blk, d_in), lambda i: (i, 0)),
            full((d_in, fh1)), full((fh1,)), full((fh1,)), full((fh1,)),
            full((fh1,)), full((fh1,)),
            full((fh1, fh2)), full((fh2,)), full((fh2,)), full((fh2,)),
            full((fh2,)), full((fh2,)),
        ],
        out_specs=pl.BlockSpec((blk, fh2), lambda i: (i, 0)),
        out_shape=jax.ShapeDtypeStruct((n, fh2), jnp.float32),
    )(x, W1, b1, g1, bb1, m1, v1, W2, b2, g2, bb2, m2, v2)


def _qkvs_out_specs(n, blk, d):
    return dict(
        out_specs=[pl.BlockSpec((blk, d), lambda i: (i, 0))] * 4,
        out_shape=[jax.ShapeDtypeStruct((n, d), jnp.float32)] * 4,
    )


def _qkvs_direct(x, Wcat, bcat, d):
    n, din = x.shape
    blk = 1024

    def body(x_ref, w_ref, b_ref, q_ref, k_ref, v_ref, s_ref):
        y = jnp.dot(x_ref[...], w_ref[...],
                    preferred_element_type=jnp.float32) + b_ref[...]
        q_ref[...] = y[:, :d]
        k_ref[...] = y[:, d:2 * d]
        v_ref[...] = y[:, 2 * d:3 * d]
        s_ref[...] = y[:, 3 * d:]

    full = lambda shape: pl.BlockSpec(shape, lambda i: (0,) * len(shape))
    return pl.pallas_call(
        body,
        grid=(n // blk,),
        in_specs=[pl.BlockSpec((blk, din), lambda i: (i, 0)),
                  full((din, 4 * d)), full((4 * d,))],
        **_qkvs_out_specs(n, blk, d),
    )(x, Wcat, bcat)


def _qkvs_mix(o2a, o2p, Wcat, bcat, d):
    _, n, din = o2a.shape
    blk = 1024

    def body(a_ref, p_ref, w_ref, b_ref, q_ref, k_ref, v_ref, s_ref):
        xin = ((1.0 - AT) * (a_ref[0] + a_ref[1])
               + AT * (p_ref[0] + p_ref[1]))
        xin = jnp.maximum(xin, 0.0)
        y = jnp.dot(xin, w_ref[...],
                    preferred_element_type=jnp.float32) + b_ref[...]
        q_ref[...] = y[:, :d]
        k_ref[...] = y[:, d:2 * d]
        v_ref[...] = y[:, 2 * d:3 * d]
        s_ref[...] = y[:, 3 * d:]

    full = lambda shape: pl.BlockSpec(shape, lambda i: (0,) * len(shape))
    return pl.pallas_call(
        body,
        grid=(n // blk,),
        in_specs=[pl.BlockSpec((NC, blk, din), lambda i: (0, i, 0)),
                  pl.BlockSpec((NC, blk, din), lambda i: (0, i, 0)),
                  full((din, 4 * d)), full((4 * d,))],
        **_qkvs_out_specs(n, blk, d),
    )(o2a, o2p, Wcat, bcat)


def _comb(den, agg, skip):
    _, n, d = agg.shape
    blk = 1024

    def body(den_ref, agg_ref, skip_ref, h_ref, g_ref):
        inv = 1.0 / (den_ref[0, :] + den_ref[1, :] + 1e-16)
        h = (agg_ref[0] + agg_ref[1]) * inv[:, None] + skip_ref[...]
        h_ref[...] = h
        g_ref[...] = h * inv[:, None]

    return pl.pallas_call(
        body,
        grid=(n // blk,),
        in_specs=[pl.BlockSpec((NC, blk), lambda i: (0, i)),
                  pl.BlockSpec((NC, blk, d), lambda i: (0, i, 0)),
                  pl.BlockSpec((blk, d), lambda i: (i, 0))],
        out_specs=[pl.BlockSpec((blk, d), lambda i: (i, 0)),
                   pl.BlockSpec((blk, d), lambda i: (i, 0))],
        out_shape=[jax.ShapeDtypeStruct((n, d), jnp.float32),
                   jax.ShapeDtypeStruct((n, d), jnp.float32)],
    )(den, agg, skip)


def _decoder(feat, codebook, o2d, o2e, dec_W, dec_b, bg, bb, bm, bv, cluster):
    n = feat.shape[0]
    blk = 512
    ncb = codebook.shape[0]
    d_out = dec_W.shape[1]

    def body(f_ref, cb_ref, a_ref, p_ref, w_ref, b_ref, g_ref, bb_ref,
             m_ref, v_ref, cl_ref, z_ref, de_ref, qd_ref):
        f = f_ref[...]
        cb = cb_ref[...]
        d2 = (jnp.sum(f * f, 1, keepdims=True)
              + jnp.sum(cb * cb, 1)[None, :]
              - 2.0 * lax.dot_general(f, cb, (((1,), (1,)), ((), ())),
                                      preferred_element_type=jnp.float32))
        mn = jnp.min(d2, axis=1, keepdims=True)
        iota = lax.broadcasted_iota(jnp.int32, d2.shape, 1)
        cand = jnp.where(d2 <= mn, iota, ncb)
        idx = jnp.min(cand, axis=1, keepdims=True)
        onehot = (iota == idx).astype(jnp.float32)
        quant = jnp.dot(onehot, cb, preferred_element_type=jnp.float32)
        mu = ((1.0 - AT) * (a_ref[0] + a_ref[1])
              + AT * (p_ref[0] + p_ref[1]))[:, :64]
        z = jnp.concatenate([quant, mu], axis=1)
        z_ref[...] = z
        de = jnp.dot(z, w_ref[...], preferred_element_type=jnp.float32)
        de_ref[...] = _bn_elu(de + b_ref[...], g_ref[...], bb_ref[...],
                              m_ref[...], v_ref[...])
        cl = cl_ref[...]
        dq = (jnp.sum(z * z, 1, keepdims=True)
              + jnp.sum(cl * cl, 1)[None, :]
              - 2.0 * lax.dot_general(z, cl, (((1,), (1,)), ((), ())),
                                      preferred_element_type=jnp.float32))
        q0 = 1.0 / (1.0 + dq)
        qd_ref[...] = q0 / jnp.sum(q0, axis=1, keepdims=True)

    full = lambda shape: pl.BlockSpec(shape, lambda i: (0,) * len(shape))
    fh2 = feat.shape[1]
    dmu = o2d.shape[2]
    ncl = cluster.shape[0]
    return pl.pallas_call(
        body,
        grid=(n // blk,),
        in_specs=[pl.BlockSpec((blk, fh2), lambda i: (i, 0)),
                  full((ncb, fh2)),
                  pl.BlockSpec((NC, blk, dmu), lambda i: (0, i, 0)),
                  pl.BlockSpec((NC, blk, dmu), lambda i: (0, i, 0)),
                  full((fh2 + 64, d_out)), full((d_out,)), full((d_out,)),
                  full((d_out,)), full((d_out,)), full((d_out,)),
                  full((ncl, fh2 + 64))],
        out_specs=[pl.BlockSpec((blk, fh2 + 64), lambda i: (i, 0)),
                   pl.BlockSpec((blk, d_out), lambda i: (i, 0)),
                   pl.BlockSpec((blk, ncl), lambda i: (i, 0))],
        out_shape=[jax.ShapeDtypeStruct((n, fh2 + 64), jnp.float32),
                   jax.ShapeDtypeStruct((n, d_out), jnp.float32),
                   jax.ShapeDtypeStruct((n, ncl), jnp.float32)],
    )(feat, codebook, o2d, o2e, dec_W, dec_b, bg, bb, bm, bv, cluster)


def kernel(x, adj, adj_prue, training,
           enc_W1, enc_b1, bn1_g, bn1_b, bn1_m, bn1_v,
           enc_W2, enc_b2, bn2_g, bn2_b, bn2_m, bn2_v,
           gc1_Wq, gc1_Wk, gc1_Wv, gc1_Ws, gc1_bq, gc1_bk, gc1_bv, gc1_bs,
           ch_Wq, ch_Wk, ch_Wv, ch_Ws, ch_bq, ch_bk, ch_bv, ch_bs,
           gc2_Wq, gc2_Wk, gc2_Wv, gc2_Ws, gc2_bq, gc2_bk, gc2_bv, gc2_bs,
           codebook, dec_W, dec_b, bnd_g, bnd_b, bnd_m, bnd_v, cluster):
    srcA, dstA = adj[0], adj[1]
    srcP, dstP = adj_prue[0], adj_prue[1]

    Wc1 = jnp.concatenate([gc1_Wq, gc1_Wk, gc1_Wv, gc1_Ws], axis=1)
    bc1 = jnp.concatenate([gc1_bq, gc1_bk, gc1_bv, gc1_bs])
    Wc2 = jnp.concatenate([ch_Wq, ch_Wk, ch_Wv, ch_Ws], axis=1)
    bc2 = jnp.concatenate([ch_bq, ch_bk, ch_bv, ch_bs])
    _pw = lambda W: jnp.pad(W, ((0, 0), (0, 64)))
    _pb = lambda b: jnp.pad(b, (0, 64))
    Wc3 = jnp.concatenate([_pw(gc2_Wq), _pw(gc2_Wk), _pw(gc2_Wv),
                           _pw(gc2_Ws)], axis=1)
    bc3 = jnp.concatenate([_pb(gc2_bq), _pb(gc2_bk), _pb(gc2_bv),
                           _pb(gc2_bs)])

    feat = _encoder(x, enc_W1, enc_b1, bn1_g, bn1_b, bn1_m, bn1_v,
                    enc_W2, enc_b2, bn2_g, bn2_b, bn2_m, bn2_v)

    _s128 = 1.0 / math.sqrt(128.0)
    _s64 = 1.0 / math.sqrt(64.0)

    q1, k1, v1, sk1 = _qkvs_direct(feat, Wc1, bc1, 128)
    ex1f = _ex_mat(q1, k1, _s128).reshape(N * N)
    exA, denA, aggA = _sc_ab(ex1f, v1, srcA, dstA, d=128)
    exP, denP, aggP = _sc_ab(ex1f, v1, srcP, dstP, d=128)
    h1, g1 = _comb(denA, aggA, sk1)
    h1p, g1p = _comb(denP, aggP, sk1)
    o2A = _sc_c(g1, srcA, dstA, exA, d=128)
    o2P = _sc_c(g1p, srcP, dstP, exP, d=128)

    q2, k2, v2, sk2 = _qkvs_mix(o2A, o2P, Wc2, bc2, 128)
    ex2f = _ex_mat(q2, k2, _s128).reshape(N * N)
    exB, denB, aggB = _sc_ab(ex2f, v2, srcA, dstA, d=128)
    x1, gB = _comb(denB, aggB, sk2)
    o2B = _sc_c(gB, srcA, dstA, exB, d=128)
    q3, k3, v3, sk3 = _qkvs_direct(x1, Wc2, bc2, 128)
    ex3f = _ex_mat(q3, k3, _s128).reshape(N * N)
    exC, denC, aggC = _sc_ab(ex3f, v3, srcP, dstP, d=128)
    xp, gC = _comb(denC, aggC, sk3)
    o2C = _sc_c(gC, srcP, dstP, exC, d=128)

    q4, k4, v4, sk4 = _qkvs_mix(o2B, o2C, Wc3, bc3, 128)
    ex4f = _ex_mat(q4, k4, _s64).reshape(N * N)
    exD, denD, aggD = _sc_ab(ex4f, v4, srcA, dstA, d=128)
    exE, denE, aggE = _sc_ab(ex4f, v4, srcP, dstP, d=128)
    muA, gD = _comb(denD, aggD, sk4)
    muP, gE = _comb(denE, aggE, sk4)
    o2D = _sc_c(gD, srcA, dstA, exD, d=128)
    o2E = _sc_c(gE, srcP, dstP, exE, d=128)

    z, de, qd = _decoder(feat, codebook, o2D, o2E, dec_W, dec_b,
                         bnd_g, bnd_b, bnd_m, bnd_v, cluster)
    return z, de, qd, feat


# speedup vs baseline: 16.4511x; 1.2150x over previous
import functools
import math

import jax
import jax.numpy as jnp
from jax import lax
from jax.experimental import pallas as pl
from jax.experimental.pallas import tpu as pltpu
from jax.experimental.pallas import tpu_sc as plsc

N = 4096
E = 65536
AT = 0.5
NC = 2
NS = 16
NW = NC * NS
L = 16
EPW = E // NW
CHK = 64
NCHUNK = EPW // CHK
ROWS = N // NS

_MESH = dict(core_axis_name="c", subcore_axis_name="s", num_cores=NC,
             num_subcores=NS)


def _zero_vmem_2d(ref, rows, d):
    zeros = jnp.zeros((L,), jnp.float32)

    def zrow(r, carry):
        for j in range(d // L):
            ref[r, pl.ds(j * L, L)] = zeros
        return carry

    lax.fori_loop(0, rows, zrow, 0)


def _zero_vmem_1d(ref, n):
    zeros = jnp.zeros((L,), jnp.float32)

    def zblk(i, carry):
        ref[pl.ds(i * L, L)] = zeros
        return carry

    lax.fori_loop(0, n // L, zblk, 0)


@functools.partial(jax.jit, static_argnames=("d",))
def _sc_ab(exf, v, src, dst, *, d):
    def body(exf_hbm, v_hbm, src_hbm, dst_hbm, ex_hbm, den_hbm, agg_hbm,
             src_v0, src_v1, dst_v0, dst_v1, dst_s0, dst_s1, exi0, exi1,
             vrows0, vrows1, srows0, srows1, exb,
             den_acc, agg_acc,
             sem_i0, sem_i1, sem_e0, sem_e1, sem_v0, sem_v1,
             sem_d0, sem_d1, sem_a0, sem_a1):
        c = lax.axis_index("c")
        s = lax.axis_index("s")
        wid = c * NS + s
        base = wid * EPW

        SRC = (src_v0, src_v1)
        DST = (dst_v0, dst_v1)
        DSC = (dst_s0, dst_s1)
        EXI = (exi0, exi1)
        VR = (vrows0, vrows1)
        SR = (srows0, srows1)
        SI = (sem_i0, sem_i1)
        SE = (sem_e0, sem_e1)
        SV = (sem_v0, sem_v1)
        SD = (sem_d0, sem_d1)
        SA = (sem_a0, sem_a1)

        _zero_vmem_2d(srows0, CHK, d)
        _zero_vmem_1d(exb, ROWS)
        for t in range(ROWS // CHK):
            pltpu.sync_copy(srows0, agg_acc.at[pl.ds(s * ROWS + t * CHK, CHK)])
        pltpu.sync_copy(exb.at[pl.ds(0, ROWS)],
                        den_acc.at[pl.ds(s * ROWS, ROWS)])
        plsc.subcore_barrier()

        def idx_issue(i, r):
            st = base + i * CHK
            pltpu.async_copy(src_hbm.at[pl.ds(st, CHK)], SRC[r], SI[r])
            pltpu.async_copy(dst_hbm.at[pl.ds(st, CHK)], DST[r], SI[r])

        def idx_wait(i, r):
            st = base + i * CHK
            pltpu.make_async_copy(src_hbm.at[pl.ds(st, CHK)], SRC[r], SI[r]).wait()
            pltpu.make_async_copy(dst_hbm.at[pl.ds(st, CHK)], DST[r], SI[r]).wait()

        def exi_compute(r):
            for t in range(CHK // L):
                sl = pl.ds(t * L, L)
                EXI[r][sl] = DST[r][sl] * N + SRC[r][sl]

        def gat_issue(i, r):
            pltpu.async_copy(exf_hbm.at[EXI[r]], exb.at[pl.ds(i * CHK, CHK)],
                             SE[r])
            pltpu.async_copy(v_hbm.at[SRC[r]], VR[r], SV[r])

        def gat_wait(i, r):
            pltpu.make_async_copy(exf_hbm.at[EXI[r]],
                                  exb.at[pl.ds(i * CHK, CHK)], SE[r]).wait()
            pltpu.make_async_copy(v_hbm.at[SRC[r]], VR[r], SV[r]).wait()

        def sca_issue(i, r):
            pltpu.async_copy(exb.at[pl.ds(i * CHK, CHK)],
                             den_acc.at[DSC[r]], SD[r], add=True)
            pltpu.async_copy(SR[r], agg_acc.at[DSC[r]], SA[r], add=True)

        def sca_wait(i, r):
            pltpu.make_async_copy(exb.at[pl.ds(i * CHK, CHK)],
                                  den_acc.at[DSC[r]], SD[r]).wait()
            pltpu.make_async_copy(SR[r], agg_acc.at[DSC[r]], SA[r]).wait()

        def snap_idx(r):
            for t in range(CHK // L):
                DSC[r][pl.ds(t * L, L)] = DST[r][pl.ds(t * L, L)]

        def scale_rows(i, r):
            vr, sr = VR[r], SR[r]

            def sgrp(g2, carry2):
                exv = exb[pl.ds(i * CHK + g2 * L, L)]
                for el in range(L):
                    sc = exv[el]
                    for j in range(d // L):
                        sr[g2 * L + el, pl.ds(j * L, L)] = (
                            vr[g2 * L + el, pl.ds(j * L, L)] * sc)
                return carry2

            lax.fori_loop(0, CHK // L, sgrp, 0)

        def one(i, r, rn):
            @pl.when(i + 1 < NCHUNK)
            def _():
                idx_wait(i + 1, rn)
                exi_compute(rn)
                gat_issue(i + 1, rn)
            gat_wait(i, r)
            snap_idx(r)
            @pl.when(i + 2 < NCHUNK)
            def _():
                idx_issue(i + 2, r)
            @pl.when(i >= 2)
            def _():
                sca_wait(i - 2, r)
            scale_rows(i, r)
            sca_issue(i, r)

        idx_issue(0, 0)
        idx_wait(0, 0)
        exi_compute(0)
        gat_issue(0, 0)
        idx_issue(1, 1)

        def pair(j, carry):
            one(2 * j, 0, 1)
            one(2 * j + 1, 1, 0)
            return carry

        lax.fori_loop(0, NCHUNK // 2, pair, 0)
        sca_wait(NCHUNK - 2, 0)
        sca_wait(NCHUNK - 1, 1)
        pltpu.sync_copy(exb, ex_hbm.at[pl.ds(base, EPW)])
        plsc.subcore_barrier()
        pltpu.sync_copy(den_acc.at[pl.ds(s * ROWS, ROWS)],
                        den_hbm.at[c, pl.ds(s * ROWS, ROWS)])
        pltpu.sync_copy(agg_acc.at[pl.ds(s * ROWS, ROWS)],
                        agg_hbm.at[c, pl.ds(s * ROWS, ROWS)])

    f = pl.kernel(
        body,
        out_type=[
            jax.ShapeDtypeStruct((E,), jnp.float32),
            jax.ShapeDtypeStruct((NC, N), jnp.float32),
            jax.ShapeDtypeStruct((NC, N, d), jnp.float32),
        ],
        mesh=plsc.VectorSubcoreMesh(**_MESH),
        compiler_params=pltpu.CompilerParams(needs_layout_passes=False),
        scratch_types=(
            [pltpu.VMEM((CHK,), jnp.int32)] * 8
            + [pltpu.VMEM((CHK, d), jnp.float32)] * 4
            + [pltpu.VMEM((EPW,), jnp.float32)]
            + [pltpu.VMEM_SHARED((N,), jnp.float32),
               pltpu.VMEM_SHARED((N, d), jnp.float32)]
            + [pltpu.SemaphoreType.DMA] * 10
        ),
    )
    return f(exf, v, src, dst)


def _ex_mat(q, k, scale):
    n, d = q.shape
    blk = 512

    def body(q_ref, k_ref, out_ref):
        y = lax.dot_general(q_ref[...], k_ref[...], (((1,), (1,)), ((), ())),
                            preferred_element_type=jnp.float32)
        out_ref[...] = jnp.exp(y * scale).reshape(blk * n)

    return pl.pallas_call(
        body,
        grid=(n // blk,),
        in_specs=[pl.BlockSpec((blk, d), lambda i: (i, 0)),
                  pl.BlockSpec((n, d), lambda i: (0, 0))],
        out_specs=pl.BlockSpec((blk * n,), lambda i: (i,)),
        out_shape=jax.ShapeDtypeStruct((n * n,), jnp.float32),
    )(q, k)


@functools.partial(jax.jit, static_argnames=("d",))
def _sc_c(g, src, dst, ex, *, d):
    def body(g_hbm, src_hbm, dst_hbm, ex_hbm, out_hbm,
             src_v0, src_v1, dst_v0, dst_v1, src_s0, src_s1,
             grows0, grows1, srows0, srows1, exb0, exb1, exk,
             acc,
             sem_i0, sem_i1, sem_g0, sem_g1, sem_a0, sem_a1):
        c = lax.axis_index("c")
        s = lax.axis_index("s")
        wid = c * NS + s
        base = wid * EPW

        SRC = (src_v0, src_v1)
        DST = (dst_v0, dst_v1)
        SSC = (src_s0, src_s1)
        GR = (grows0, grows1)
        SR = (srows0, srows1)
        EXB = (exb0, exb1)
        SI = (sem_i0, sem_i1)
        SG = (sem_g0, sem_g1)
        SA = (sem_a0, sem_a1)

        _zero_vmem_2d(srows0, CHK, d)
        for t in range(ROWS // CHK):
            pltpu.sync_copy(srows0, acc.at[pl.ds(s * ROWS + t * CHK, CHK)])
        plsc.subcore_barrier()

        def idx_issue(i, r):
            st = base + i * CHK
            pltpu.async_copy(src_hbm.at[pl.ds(st, CHK)], SRC[r], SI[r])
            pltpu.async_copy(dst_hbm.at[pl.ds(st, CHK)], DST[r], SI[r])
            pltpu.async_copy(ex_hbm.at[pl.ds(st, CHK)], EXB[r], SI[r])

        def idx_wait(i, r):
            st = base + i * CHK
            pltpu.make_async_copy(src_hbm.at[pl.ds(st, CHK)], SRC[r], SI[r]).wait()
            pltpu.make_async_copy(dst_hbm.at[pl.ds(st, CHK)], DST[r], SI[r]).wait()
            pltpu.make_async_copy(ex_hbm.at[pl.ds(st, CHK)], EXB[r], SI[r]).wait()

        def gat_issue(r):
            pltpu.async_copy(g_hbm.at[DST[r]], GR[r], SG[r])

        def gat_wait(r):
            pltpu.make_async_copy(g_hbm.at[DST[r]], GR[r], SG[r]).wait()

        def sca_issue(r):
            pltpu.async_copy(SR[r], acc.at[SSC[r]], SA[r], add=True)

        def sca_wait(r):
            pltpu.make_async_copy(SR[r], acc.at[SSC[r]], SA[r]).wait()

        def snap_idx(r):
            for t in range(CHK // L):
                sl = pl.ds(t * L, L)
                SSC[r][sl] = SRC[r][sl]
                keep = SRC[r][sl] != DST[r][sl]
                exk[sl] = jnp.where(keep, EXB[r][sl], 0.0)

        def scale_rows(r):
            def sgrp(g2, carry2):
                exv = exk[pl.ds(g2 * L, L)]
                for el in range(L):
                    sc = exv[el]
                    for j in range(d // L):
                        SR[r][g2 * L + el, pl.ds(j * L, L)] = (
                            GR[r][g2 * L + el, pl.ds(j * L, L)] * sc)
                return carry2

            lax.fori_loop(0, CHK // L, sgrp, 0)

        def one(i, r, rn):
            @pl.when(i + 1 < NCHUNK)
            def _():
                idx_wait(i + 1, rn)
                gat_issue(rn)
            gat_wait(r)
            snap_idx(r)
            @pl.when(i + 2 < NCHUNK)
            def _():
                idx_issue(i + 2, r)
            @pl.when(i >= 2)
            def _():
                sca_wait(r)
            scale_rows(r)
            sca_issue(r)

        idx_issue(0, 0)
        idx_wait(0, 0)
        gat_issue(0)
        idx_issue(1, 1)

        def pair(j, carry):
            one(2 * j, 0, 1)
            one(2 * j + 1, 1, 0)
            return carry

        lax.fori_loop(0, NCHUNK // 2, pair, 0)
        sca_wait(0)
        sca_wait(1)
        plsc.subcore_barrier()
        pltpu.sync_copy(acc.at[pl.ds(s * ROWS, ROWS)],
                        out_hbm.at[c, pl.ds(s * ROWS, ROWS)])

    f = pl.kernel(
        body,
        out_type=[jax.ShapeDtypeStruct((NC, N, d), jnp.float32)],
        mesh=plsc.VectorSubcoreMesh(**_MESH),
        compiler_params=pltpu.CompilerParams(needs_layout_passes=False),
        scratch_types=(
            [pltpu.VMEM((CHK,), jnp.int32)] * 6
            + [pltpu.VMEM((CHK, d), jnp.float32)] * 4
            + [pltpu.VMEM((CHK,), jnp.float32)] * 3
            + [pltpu.VMEM_SHARED((N, d), jnp.float32)]
            + [pltpu.SemaphoreType.DMA] * 6
        ),
    )
    return f(g, src, dst, ex)[0]


def _bn_elu(y, g, b, m, v):
    y = g * (y - m) * lax.rsqrt(v + 1e-4) + b
    return jnp.where(y > 0, y, jnp.exp(y) - 1.0)


def _enc_body(x_ref, W1_ref, b1_ref, g1_ref, bb1_ref, m1_ref, v1_ref,
              W2_ref, b2_ref, g2_ref, bb2_ref, m2_ref, v2_ref, out_ref):
    y = jnp.dot(x_ref[...], W1_ref[...], preferred_element_type=jnp.float32)
    y = _bn_elu(y + b1_ref[...], g1_ref[...], bb1_ref[...], m1_ref[...],
                v1_ref[...])
    z = jnp.dot(y, W2_ref[...], preferred_element_type=jnp.float32)
    z = _bn_elu(z + b2_ref[...], g2_ref[...], bb2_ref[...], m2_ref[...],
                v2_ref[...])
    out_ref[...] = z


def _encoder(x, W1, b1, g1, bb1, m1, v1, W2, b2, g2, bb2, m2, v2):
    n, d_in = x.shape
    fh1 = W1.shape[1]
    fh2 = W2.shape[1]
    blk = 512
    full = lambda shape: pl.BlockSpec(shape, lambda i: (0,) * len(shape))
    return pl.pallas_call(
        _enc_body,
        grid=(n // blk,),
        in_specs=[
            pl.BlockSpec((blk, d_in), lambda i: (i, 0)),
            full((d_in, fh1)), full((fh1,)), full((fh1,)), full((fh1,)),
            full((fh1,)), full((fh1,)),
            full((fh1, fh2)), full((fh2,)), full((fh2,)), full((fh2,)),
            full((fh2,)), full((fh2,)),
        ],
        out_specs=pl.BlockSpec((blk, fh2), lambda i: (i, 0)),
        out_shape=jax.ShapeDtypeStruct((n, fh2), jnp.float32),
    )(x, W1, b1, g1, bb1, m1, v1, W2, b2, g2, bb2, m2, v2)


def _qkvs_out_specs(n, blk, d):
    return dict(
        out_specs=[pl.BlockSpec((blk, d), lambda i: (i, 0))] * 4,
        out_shape=[jax.ShapeDtypeStruct((n, d), jnp.float32)] * 4,
    )


def _qkvs_direct(x, Wcat, bcat, d):
    n, din = x.shape
    blk = 1024

    def body(x_ref, w_ref, b_ref, q_ref, k_ref, v_ref, s_ref):
        y = jnp.dot(x_ref[...], w_ref[...],
                    preferred_element_type=jnp.float32) + b_ref[...]
        q_ref[...] = y[:, :d]
        k_ref[...] = y[:, d:2 * d]
        v_ref[...] = y[:, 2 * d:3 * d]
        s_ref[...] = y[:, 3 * d:]

    full = lambda shape: pl.BlockSpec(shape, lambda i: (0,) * len(shape))
    return pl.pallas_call(
        body,
        grid=(n // blk,),
        in_specs=[pl.BlockSpec((blk, din), lambda i: (i, 0)),
                  full((din, 4 * d)), full((4 * d,))],
        **_qkvs_out_specs(n, blk, d),
    )(x, Wcat, bcat)


def _qkvs_mix(o2a, o2p, Wcat, bcat, d):
    _, n, din = o2a.shape
    blk = 1024

    def body(a_ref, p_ref, w_ref, b_ref, q_ref, k_ref, v_ref, s_ref):
        xin = ((1.0 - AT) * (a_ref[0] + a_ref[1])
               + AT * (p_ref[0] + p_ref[1]))
        xin = jnp.maximum(xin, 0.0)
        y = jnp.dot(xin, w_ref[...],
                    preferred_element_type=jnp.float32) + b_ref[...]
        q_ref[...] = y[:, :d]
        k_ref[...] = y[:, d:2 * d]
        v_ref[...] = y[:, 2 * d:3 * d]
        s_ref[...] = y[:, 3 * d:]

    full = lambda shape: pl.BlockSpec(shape, lambda i: (0,) * len(shape))
    return pl.pallas_call(
        body,
        grid=(n // blk,),
        in_specs=[pl.BlockSpec((NC, blk, din), lambda i: (0, i, 0)),
                  pl.BlockSpec((NC, blk, din), lambda i: (0, i, 0)),
                  full((din, 4 * d)), full((4 * d,))],
        **_qkvs_out_specs(n, blk, d),
    )(o2a, o2p, Wcat, bcat)


def _comb(den, agg, skip):
    _, n, d = agg.shape
    blk = 1024

    def body(den_ref, agg_ref, skip_ref, h_ref, g_ref):
        inv = 1.0 / (den_ref[0, :] + den_ref[1, :] + 1e-16)
        h = (agg_ref[0] + agg_ref[1]) * inv[:, None] + skip_ref[...]
        h_ref[...] = h
        g_ref[...] = h * inv[:, None]

    return pl.pallas_call(
        body,
        grid=(n // blk,),
        in_specs=[pl.BlockSpec((NC, blk), lambda i: (0, i)),
                  pl.BlockSpec((NC, blk, d), lambda i: (0, i, 0)),
                  pl.BlockSpec((blk, d), lambda i: (i, 0))],
        out_specs=[pl.BlockSpec((blk, d), lambda i: (i, 0)),
                   pl.BlockSpec((blk, d), lambda i: (i, 0))],
        out_shape=[jax.ShapeDtypeStruct((n, d), jnp.float32),
                   jax.ShapeDtypeStruct((n, d), jnp.float32)],
    )(den, agg, skip)


def _decoder(feat, codebook, o2d, o2e, dec_W, dec_b, bg, bb, bm, bv, cluster):
    n = feat.shape[0]
    blk = 512
    ncb = codebook.shape[0]
    d_out = dec_W.shape[1]

    def body(f_ref, cb_ref, a_ref, p_ref, w_ref, b_ref, g_ref, bb_ref,
             m_ref, v_ref, cl_ref, z_ref, de_ref, qd_ref):
        f = f_ref[...]
        cb = cb_ref[...]
        d2 = (jnp.sum(f * f, 1, keepdims=True)
              + jnp.sum(cb * cb, 1)[None, :]
              - 2.0 * lax.dot_general(f, cb, (((1,), (1,)), ((), ())),
                                      preferred_element_type=jnp.float32))
        mn = jnp.min(d2, axis=1, keepdims=True)
        iota = lax.broadcasted_iota(jnp.int32, d2.shape, 1)
        cand = jnp.where(d2 <= mn, iota, ncb)
        idx = jnp.min(cand, axis=1, keepdims=True)
        onehot = (iota == idx).astype(jnp.float32)
        quant = jnp.dot(onehot, cb, preferred_element_type=jnp.float32)
        mu = ((1.0 - AT) * (a_ref[0] + a_ref[1])
              + AT * (p_ref[0] + p_ref[1]))[:, :64]
        z = jnp.concatenate([quant, mu], axis=1)
        z_ref[...] = z
        de = jnp.dot(z, w_ref[...], preferred_element_type=jnp.float32)
        de_ref[...] = _bn_elu(de + b_ref[...], g_ref[...], bb_ref[...],
                              m_ref[...], v_ref[...])
        cl = cl_ref[...]
        dq = (jnp.sum(z * z, 1, keepdims=True)
              + jnp.sum(cl * cl, 1)[None, :]
              - 2.0 * lax.dot_general(z, cl, (((1,), (1,)), ((), ())),
                                      preferred_element_type=jnp.float32))
        q0 = 1.0 / (1.0 + dq)
        qd_ref[...] = q0 / jnp.sum(q0, axis=1, keepdims=True)

    full = lambda shape: pl.BlockSpec(shape, lambda i: (0,) * len(shape))
    fh2 = feat.shape[1]
    dmu = o2d.shape[2]
    ncl = cluster.shape[0]
    return pl.pallas_call(
        body,
        grid=(n // blk,),
        in_specs=[pl.BlockSpec((blk, fh2), lambda i: (i, 0)),
                  full((ncb, fh2)),
                  pl.BlockSpec((NC, blk, dmu), lambda i: (0, i, 0)),
                  pl.BlockSpec((NC, blk, dmu), lambda i: (0, i, 0)),
                  full((fh2 + 64, d_out)), full((d_out,)), full((d_out,)),
                  full((d_out,)), full((d_out,)), full((d_out,)),
                  full((ncl, fh2 + 64))],
        out_specs=[pl.BlockSpec((blk, fh2 + 64), lambda i: (i, 0)),
                   pl.BlockSpec((blk, d_out), lambda i: (i, 0)),
                   pl.BlockSpec((blk, ncl), lambda i: (i, 0))],
        out_shape=[jax.ShapeDtypeStruct((n, fh2 + 64), jnp.float32),
                   jax.ShapeDtypeStruct((n, d_out), jnp.float32),
                   jax.ShapeDtypeStruct((n, ncl), jnp.float32)],
    )(feat, codebook, o2d, o2e, dec_W, dec_b, bg, bb, bm, bv, cluster)


def kernel(x, adj, adj_prue, training,
           enc_W1, enc_b1, bn1_g, bn1_b, bn1_m, bn1_v,
           enc_W2, enc_b2, bn2_g, bn2_b, bn2_m, bn2_v,
           gc1_Wq, gc1_Wk, gc1_Wv, gc1_Ws, gc1_bq, gc1_bk, gc1_bv, gc1_bs,
           ch_Wq, ch_Wk, ch_Wv, ch_Ws, ch_bq, ch_bk, ch_bv, ch_bs,
           gc2_Wq, gc2_Wk, gc2_Wv, gc2_Ws, gc2_bq, gc2_bk, gc2_bv, gc2_bs,
           codebook, dec_W, dec_b, bnd_g, bnd_b, bnd_m, bnd_v, cluster):
    srcA, dstA = adj[0], adj[1]
    srcP, dstP = adj_prue[0], adj_prue[1]

    Wc1 = jnp.concatenate([gc1_Wq, gc1_Wk, gc1_Wv, gc1_Ws], axis=1)
    bc1 = jnp.concatenate([gc1_bq, gc1_bk, gc1_bv, gc1_bs])
    Wc2 = jnp.concatenate([ch_Wq, ch_Wk, ch_Wv, ch_Ws], axis=1)
    bc2 = jnp.concatenate([ch_bq, ch_bk, ch_bv, ch_bs])
    _pw = lambda W: jnp.pad(W, ((0, 0), (0, 64)))
    _pb = lambda b: jnp.pad(b, (0, 64))
    Wc3 = jnp.concatenate([_pw(gc2_Wq), _pw(gc2_Wk), _pw(gc2_Wv),
                           _pw(gc2_Ws)], axis=1)
    bc3 = jnp.concatenate([_pb(gc2_bq), _pb(gc2_bk), _pb(gc2_bv),
                           _pb(gc2_bs)])

    feat = _encoder(x, enc_W1, enc_b1, bn1_g, bn1_b, bn1_m, bn1_v,
                    enc_W2, enc_b2, bn2_g, bn2_b, bn2_m, bn2_v)

    _s128 = 1.0 / math.sqrt(128.0)
    _s64 = 1.0 / math.sqrt(64.0)

    q1, k1, v1, sk1 = _qkvs_direct(feat, Wc1, bc1, 128)
    ex1f = _ex_mat(q1, k1, _s128)
    exA, denA, aggA = _sc_ab(ex1f, v1, srcA, dstA, d=128)
    exP, denP, aggP = _sc_ab(ex1f, v1, srcP, dstP, d=128)
    h1, g1 = _comb(denA, aggA, sk1)
    h1p, g1p = _comb(denP, aggP, sk1)
    o2A = _sc_c(g1, srcA, dstA, exA, d=128)
    o2P = _sc_c(g1p, srcP, dstP, exP, d=128)

    q2, k2, v2, sk2 = _qkvs_mix(o2A, o2P, Wc2, bc2, 128)
    ex2f = _ex_mat(q2, k2, _s128)
    exB, denB, aggB = _sc_ab(ex2f, v2, srcA, dstA, d=128)
    x1, gB = _comb(denB, aggB, sk2)
    o2B = _sc_c(gB, srcA, dstA, exB, d=128)
    q3, k3, v3, sk3 = _qkvs_direct(x1, Wc2, bc2, 128)
    ex3f = _ex_mat(q3, k3, _s128)
    exC, denC, aggC = _sc_ab(ex3f, v3, srcP, dstP, d=128)
    xp, gC = _comb(denC, aggC, sk3)
    o2C = _sc_c(gC, srcP, dstP, exC, d=128)

    q4, k4, v4, sk4 = _qkvs_mix(o2B, o2C, Wc3, bc3, 128)
    ex4f = _ex_mat(q4, k4, _s64)
    exD, denD, aggD = _sc_ab(ex4f, v4, srcA, dstA, d=128)
    exE, denE, aggE = _sc_ab(ex4f, v4, srcP, dstP, d=128)
    muA, gD = _comb(denD, aggD, sk4)
    muP, gE = _comb(denE, aggE, sk4)
    o2D = _sc_c(gD, srcA, dstA, exD, d=128)
    o2E = _sc_c(gE, srcP, dstP, exE, d=128)

    z, de, qd = _decoder(feat, codebook, o2D, o2E, dec_W, dec_b,
                         bnd_g, bnd_b, bnd_m, bnd_v, cluster)
    return z, de, qd, feat


# speedup vs baseline: 17.0384x; 1.0357x over previous
import functools
import math

import jax
import jax.numpy as jnp
from jax import lax
from jax.experimental import pallas as pl
from jax.experimental.pallas import tpu as pltpu
from jax.experimental.pallas import tpu_sc as plsc

N = 4096
E = 65536
AT = 0.5
NC = 2
NS = 16
NW = NC * NS
L = 16
EPW = E // NW
CHK = 128
NCHUNK = EPW // CHK
ROWS = N // NS

_MESH = dict(core_axis_name="c", subcore_axis_name="s", num_cores=NC,
             num_subcores=NS)


def _zero_vmem_2d(ref, rows, d):
    zeros = jnp.zeros((L,), jnp.float32)

    def zrow(r, carry):
        for j in range(d // L):
            ref[r, pl.ds(j * L, L)] = zeros
        return carry

    lax.fori_loop(0, rows, zrow, 0)


def _zero_vmem_1d(ref, n):
    zeros = jnp.zeros((L,), jnp.float32)

    def zblk(i, carry):
        ref[pl.ds(i * L, L)] = zeros
        return carry

    lax.fori_loop(0, n // L, zblk, 0)


@functools.partial(jax.jit, static_argnames=("d",))
def _sc_ab(exf, v, src, dst, *, d):
    def body(exf_hbm, v_hbm, src_hbm, dst_hbm, ex_hbm, den_hbm, agg_hbm,
             src_v0, src_v1, dst_v0, dst_v1, dst_s0, dst_s1, exi0, exi1,
             vrows0, vrows1, srows0, srows1, exb,
             den_acc, agg_acc,
             sem_i0, sem_i1, sem_e0, sem_e1, sem_v0, sem_v1,
             sem_d0, sem_d1, sem_a0, sem_a1):
        c = lax.axis_index("c")
        s = lax.axis_index("s")
        wid = c * NS + s
        base = wid * EPW

        SRC = (src_v0, src_v1)
        DST = (dst_v0, dst_v1)
        DSC = (dst_s0, dst_s1)
        EXI = (exi0, exi1)
        VR = (vrows0, vrows1)
        SR = (srows0, srows1)
        SI = (sem_i0, sem_i1)
        SE = (sem_e0, sem_e1)
        SV = (sem_v0, sem_v1)
        SD = (sem_d0, sem_d1)
        SA = (sem_a0, sem_a1)

        _zero_vmem_2d(srows0, CHK, d)
        _zero_vmem_1d(exb, ROWS)
        for t in range(ROWS // CHK):
            pltpu.sync_copy(srows0, agg_acc.at[pl.ds(s * ROWS + t * CHK, CHK)])
        pltpu.sync_copy(exb.at[pl.ds(0, ROWS)],
                        den_acc.at[pl.ds(s * ROWS, ROWS)])
        plsc.subcore_barrier()

        def idx_issue(i, r):
            st = base + i * CHK
            pltpu.async_copy(src_hbm.at[pl.ds(st, CHK)], SRC[r], SI[r])
            pltpu.async_copy(dst_hbm.at[pl.ds(st, CHK)], DST[r], SI[r])

        def idx_wait(i, r):
            st = base + i * CHK
            pltpu.make_async_copy(src_hbm.at[pl.ds(st, CHK)], SRC[r], SI[r]).wait()
            pltpu.make_async_copy(dst_hbm.at[pl.ds(st, CHK)], DST[r], SI[r]).wait()

        def exi_compute(r):
            for t in range(CHK // L):
                sl = pl.ds(t * L, L)
                EXI[r][sl] = DST[r][sl] * N + SRC[r][sl]

        def gat_issue(i, r):
            pltpu.async_copy(exf_hbm.at[EXI[r]], exb.at[pl.ds(i * CHK, CHK)],
                             SE[r])
            pltpu.async_copy(v_hbm.at[SRC[r]], VR[r], SV[r])

        def gat_wait(i, r):
            pltpu.make_async_copy(exf_hbm.at[EXI[r]],
                                  exb.at[pl.ds(i * CHK, CHK)], SE[r]).wait()
            pltpu.make_async_copy(v_hbm.at[SRC[r]], VR[r], SV[r]).wait()

        def sca_issue(i, r):
            pltpu.async_copy(exb.at[pl.ds(i * CHK, CHK)],
                             den_acc.at[DSC[r]], SD[r], add=True)
            pltpu.async_copy(SR[r], agg_acc.at[DSC[r]], SA[r], add=True)

        def sca_wait(i, r):
            pltpu.make_async_copy(exb.at[pl.ds(i * CHK, CHK)],
                                  den_acc.at[DSC[r]], SD[r]).wait()
            pltpu.make_async_copy(SR[r], agg_acc.at[DSC[r]], SA[r]).wait()

        def snap_idx(r):
            for t in range(CHK // L):
                DSC[r][pl.ds(t * L, L)] = DST[r][pl.ds(t * L, L)]

        def scale_rows(i, r):
            vr, sr = VR[r], SR[r]

            def sgrp(g2, carry2):
                exv = exb[pl.ds(i * CHK + g2 * L, L)]
                for el in range(L):
                    sc = exv[el]
                    for j in range(d // L):
                        sr[g2 * L + el, pl.ds(j * L, L)] = (
                            vr[g2 * L + el, pl.ds(j * L, L)] * sc)
                return carry2

            lax.fori_loop(0, CHK // L, sgrp, 0)

        def one(i, r, rn):
            @pl.when(i + 1 < NCHUNK)
            def _():
                idx_wait(i + 1, rn)
                exi_compute(rn)
                gat_issue(i + 1, rn)
            gat_wait(i, r)
            snap_idx(r)
            @pl.when(i + 2 < NCHUNK)
            def _():
                idx_issue(i + 2, r)
            @pl.when(i >= 2)
            def _():
                sca_wait(i - 2, r)
            scale_rows(i, r)
            sca_issue(i, r)

        idx_issue(0, 0)
        idx_wait(0, 0)
        exi_compute(0)
        gat_issue(0, 0)
        idx_issue(1, 1)

        def pair(j, carry):
            one(2 * j, 0, 1)
            one(2 * j + 1, 1, 0)
            return carry

        lax.fori_loop(0, NCHUNK // 2, pair, 0)
        sca_wait(NCHUNK - 2, 0)
        sca_wait(NCHUNK - 1, 1)
        pltpu.sync_copy(exb, ex_hbm.at[pl.ds(base, EPW)])
        plsc.subcore_barrier()
        pltpu.sync_copy(den_acc.at[pl.ds(s * ROWS, ROWS)],
                        den_hbm.at[c, pl.ds(s * ROWS, ROWS)])
        pltpu.sync_copy(agg_acc.at[pl.ds(s * ROWS, ROWS)],
                        agg_hbm.at[c, pl.ds(s * ROWS, ROWS)])

    f = pl.kernel(
        body,
        out_type=[
            jax.ShapeDtypeStruct((E,), jnp.float32),
            jax.ShapeDtypeStruct((NC, N), jnp.float32),
            jax.ShapeDtypeStruct((NC, N, d), jnp.float32),
        ],
        mesh=plsc.VectorSubcoreMesh(**_MESH),
        compiler_params=pltpu.CompilerParams(needs_layout_passes=False),
        scratch_types=(
            [pltpu.VMEM((CHK,), jnp.int32)] * 8
            + [pltpu.VMEM((CHK, d), jnp.float32)] * 4
            + [pltpu.VMEM((EPW,), jnp.float32)]
            + [pltpu.VMEM_SHARED((N,), jnp.float32),
               pltpu.VMEM_SHARED((N, d), jnp.float32)]
            + [pltpu.SemaphoreType.DMA] * 10
        ),
    )
    return f(exf, v, src, dst)


def _ex_mat(q, k, scale):
    n, d = q.shape
    blk = 512

    def body(q_ref, k_ref, out_ref):
        y = lax.dot_general(q_ref[...], k_ref[...], (((1,), (1,)), ((), ())),
                            preferred_element_type=jnp.float32)
        out_ref[...] = jnp.exp(y * scale).reshape(blk * n)

    return pl.pallas_call(
        body,
        grid=(n // blk,),
        in_specs=[pl.BlockSpec((blk, d), lambda i: (i, 0)),
                  pl.BlockSpec((n, d), lambda i: (0, 0))],
        out_specs=pl.BlockSpec((blk * n,), lambda i: (i,)),
        out_shape=jax.ShapeDtypeStruct((n * n,), jnp.float32),
    )(q, k)


@functools.partial(jax.jit, static_argnames=("d",))
def _sc_c(g, src, dst, ex, *, d):
    def body(g_hbm, src_hbm, dst_hbm, ex_hbm, out_hbm,
             src_v0, src_v1, dst_v0, dst_v1, src_s0, src_s1,
             grows0, grows1, srows0, srows1, exb0, exb1, exk,
             acc,
             sem_i0, sem_i1, sem_g0, sem_g1, sem_a0, sem_a1):
        c = lax.axis_index("c")
        s = lax.axis_index("s")
        wid = c * NS + s
        base = wid * EPW

        SRC = (src_v0, src_v1)
        DST = (dst_v0, dst_v1)
        SSC = (src_s0, src_s1)
        GR = (grows0, grows1)
        SR = (srows0, srows1)
        EXB = (exb0, exb1)
        SI = (sem_i0, sem_i1)
        SG = (sem_g0, sem_g1)
        SA = (sem_a0, sem_a1)

        _zero_vmem_2d(srows0, CHK, d)
        for t in range(ROWS // CHK):
            pltpu.sync_copy(srows0, acc.at[pl.ds(s * ROWS + t * CHK, CHK)])
        plsc.subcore_barrier()

        def idx_issue(i, r):
            st = base + i * CHK
            pltpu.async_copy(src_hbm.at[pl.ds(st, CHK)], SRC[r], SI[r])
            pltpu.async_copy(dst_hbm.at[pl.ds(st, CHK)], DST[r], SI[r])
            pltpu.async_copy(ex_hbm.at[pl.ds(st, CHK)], EXB[r], SI[r])

        def idx_wait(i, r):
            st = base + i * CHK
            pltpu.make_async_copy(src_hbm.at[pl.ds(st, CHK)], SRC[r], SI[r]).wait()
            pltpu.make_async_copy(dst_hbm.at[pl.ds(st, CHK)], DST[r], SI[r]).wait()
            pltpu.make_async_copy(ex_hbm.at[pl.ds(st, CHK)], EXB[r], SI[r]).wait()

        def gat_issue(r):
            pltpu.async_copy(g_hbm.at[DST[r]], GR[r], SG[r])

        def gat_wait(r):
            pltpu.make_async_copy(g_hbm.at[DST[r]], GR[r], SG[r]).wait()

        def sca_issue(r):
            pltpu.async_copy(SR[r], acc.at[SSC[r]], SA[r], add=True)

        def sca_wait(r):
            pltpu.make_async_copy(SR[r], acc.at[SSC[r]], SA[r]).wait()

        def snap_idx(r):
            for t in range(CHK // L):
                sl = pl.ds(t * L, L)
                SSC[r][sl] = SRC[r][sl]
                keep = SRC[r][sl] != DST[r][sl]
                exk[sl] = jnp.where(keep, EXB[r][sl], 0.0)

        def scale_rows(r):
            def sgrp(g2, carry2):
                exv = exk[pl.ds(g2 * L, L)]
                for el in range(L):
                    sc = exv[el]
                    for j in range(d // L):
                        SR[r][g2 * L + el, pl.ds(j * L, L)] = (
                            GR[r][g2 * L + el, pl.ds(j * L, L)] * sc)
                return carry2

            lax.fori_loop(0, CHK // L, sgrp, 0)

        def one(i, r, rn):
            @pl.when(i + 1 < NCHUNK)
            def _():
                idx_wait(i + 1, rn)
                gat_issue(rn)
            gat_wait(r)
            snap_idx(r)
            @pl.when(i + 2 < NCHUNK)
            def _():
                idx_issue(i + 2, r)
            @pl.when(i >= 2)
            def _():
                sca_wait(r)
            scale_rows(r)
            sca_issue(r)

        idx_issue(0, 0)
        idx_wait(0, 0)
        gat_issue(0)
        idx_issue(1, 1)

        def pair(j, carry):
            one(2 * j, 0, 1)
            one(2 * j + 1, 1, 0)
            return carry

        lax.fori_loop(0, NCHUNK // 2, pair, 0)
        sca_wait(0)
        sca_wait(1)
        plsc.subcore_barrier()
        pltpu.sync_copy(acc.at[pl.ds(s * ROWS, ROWS)],
                        out_hbm.at[c, pl.ds(s * ROWS, ROWS)])

    f = pl.kernel(
        body,
        out_type=[jax.ShapeDtypeStruct((NC, N, d), jnp.float32)],
        mesh=plsc.VectorSubcoreMesh(**_MESH),
        compiler_params=pltpu.CompilerParams(needs_layout_passes=False),
        scratch_types=(
            [pltpu.VMEM((CHK,), jnp.int32)] * 6
            + [pltpu.VMEM((CHK, d), jnp.float32)] * 4
            + [pltpu.VMEM((CHK,), jnp.float32)] * 3
            + [pltpu.VMEM_SHARED((N, d), jnp.float32)]
            + [pltpu.SemaphoreType.DMA] * 6
        ),
    )
    return f(g, src, dst, ex)[0]


def _bn_elu(y, g, b, m, v):
    y = g * (y - m) * lax.rsqrt(v + 1e-4) + b
    return jnp.where(y > 0, y, jnp.exp(y) - 1.0)


def _enc_body(x_ref, W1_ref, b1_ref, g1_ref, bb1_ref, m1_ref, v1_ref,
              W2_ref, b2_ref, g2_ref, bb2_ref, m2_ref, v2_ref, out_ref):
    y = jnp.dot(x_ref[...], W1_ref[...], preferred_element_type=jnp.float32)
    y = _bn_elu(y + b1_ref[...], g1_ref[...], bb1_ref[...], m1_ref[...],
                v1_ref[...])
    z = jnp.dot(y, W2_ref[...], preferred_element_type=jnp.float32)
    z = _bn_elu(z + b2_ref[...], g2_ref[...], bb2_ref[...], m2_ref[...],
                v2_ref[...])
    out_ref[...] = z


def _encoder(x, W1, b1, g1, bb1, m1, v1, W2, b2, g2, bb2, m2, v2):
    n, d_in = x.shape
    fh1 = W1.shape[1]
    fh2 = W2.shape[1]
    blk = 512
    full = lambda shape: pl.BlockSpec(shape, lambda i: (0,) * len(shape))
    return pl.pallas_call(
        _enc_body,
        grid=(n // blk,),
        in_specs=[
            pl.BlockSpec((blk, d_in), lambda i: (i, 0)),
            full((d_in, fh1)), full((fh1,)), full((fh1,)), full((fh1,)),
            full((fh1,)), full((fh1,)),
            full((fh1, fh2)), full((fh2,)), full((fh2,)), full((fh2,)),
            full((fh2,)), full((fh2,)),
        ],
        out_specs=pl.BlockSpec((blk, fh2), lambda i: (i, 0)),
        out_shape=jax.ShapeDtypeStruct((n, fh2), jnp.float32),
    )(x, W1, b1, g1, bb1, m1, v1, W2, b2, g2, bb2, m2, v2)


def _qkvs_out_specs(n, blk, d):
    return dict(
        out_specs=[pl.BlockSpec((blk, d), lambda i: (i, 0))] * 4,
        out_shape=[jax.ShapeDtypeStruct((n, d), jnp.float32)] * 4,
    )


def _qkvs_direct(x, Wcat, bcat, d):
    n, din = x.shape
    blk = 1024

    def body(x_ref, w_ref, b_ref, q_ref, k_ref, v_ref, s_ref):
        y = jnp.dot(x_ref[...], w_ref[...],
                    preferred_element_type=jnp.float32) + b_ref[...]
        q_ref[...] = y[:, :d]
        k_ref[...] = y[:, d:2 * d]
        v_ref[...] = y[:, 2 * d:3 * d]
        s_ref[...] = y[:, 3 * d:]

    full = lambda shape: pl.BlockSpec(shape, lambda i: (0,) * len(shape))
    return pl.pallas_call(
        body,
        grid=(n // blk,),
        in_specs=[pl.BlockSpec((blk, din), lambda i: (i, 0)),
                  full((din, 4 * d)), full((4 * d,))],
        **_qkvs_out_specs(n, blk, d),
    )(x, Wcat, bcat)


def _qkvs_mix(o2a, o2p, Wcat, bcat, d):
    _, n, din = o2a.shape
    blk = 1024

    def body(a_ref, p_ref, w_ref, b_ref, q_ref, k_ref, v_ref, s_ref):
        xin = ((1.0 - AT) * (a_ref[0] + a_ref[1])
               + AT * (p_ref[0] + p_ref[1]))
        xin = jnp.maximum(xin, 0.0)
        y = jnp.dot(xin, w_ref[...],
                    preferred_element_type=jnp.float32) + b_ref[...]
        q_ref[...] = y[:, :d]
        k_ref[...] = y[:, d:2 * d]
        v_ref[...] = y[:, 2 * d:3 * d]
        s_ref[...] = y[:, 3 * d:]

    full = lambda shape: pl.BlockSpec(shape, lambda i: (0,) * len(shape))
    return pl.pallas_call(
        body,
        grid=(n // blk,),
        in_specs=[pl.BlockSpec((NC, blk, din), lambda i: (0, i, 0)),
                  pl.BlockSpec((NC, blk, din), lambda i: (0, i, 0)),
                  full((din, 4 * d)), full((4 * d,))],
        **_qkvs_out_specs(n, blk, d),
    )(o2a, o2p, Wcat, bcat)


def _comb(den, agg, skip):
    _, n, d = agg.shape
    blk = 1024

    def body(den_ref, agg_ref, skip_ref, h_ref, g_ref):
        inv = 1.0 / (den_ref[0, :] + den_ref[1, :] + 1e-16)
        h = (agg_ref[0] + agg_ref[1]) * inv[:, None] + skip_ref[...]
        h_ref[...] = h
        g_ref[...] = h * inv[:, None]

    return pl.pallas_call(
        body,
        grid=(n // blk,),
        in_specs=[pl.BlockSpec((NC, blk), lambda i: (0, i)),
                  pl.BlockSpec((NC, blk, d), lambda i: (0, i, 0)),
                  pl.BlockSpec((blk, d), lambda i: (i, 0))],
        out_specs=[pl.BlockSpec((blk, d), lambda i: (i, 0)),
                   pl.BlockSpec((blk, d), lambda i: (i, 0))],
        out_shape=[jax.ShapeDtypeStruct((n, d), jnp.float32),
                   jax.ShapeDtypeStruct((n, d), jnp.float32)],
    )(den, agg, skip)


def _decoder(feat, codebook, o2d, o2e, dec_W, dec_b, bg, bb, bm, bv, cluster):
    n = feat.shape[0]
    blk = 512
    ncb = codebook.shape[0]
    d_out = dec_W.shape[1]

    def body(f_ref, cb_ref, a_ref, p_ref, w_ref, b_ref, g_ref, bb_ref,
             m_ref, v_ref, cl_ref, z_ref, de_ref, qd_ref):
        f = f_ref[...]
        cb = cb_ref[...]
        d2 = (jnp.sum(f * f, 1, keepdims=True)
              + jnp.sum(cb * cb, 1)[None, :]
              - 2.0 * lax.dot_general(f, cb, (((1,), (1,)), ((), ())),
                                      preferred_element_type=jnp.float32))
        mn = jnp.min(d2, axis=1, keepdims=True)
        iota = lax.broadcasted_iota(jnp.int32, d2.shape, 1)
        cand = jnp.where(d2 <= mn, iota, ncb)
        idx = jnp.min(cand, axis=1, keepdims=True)
        onehot = (iota == idx).astype(jnp.float32)
        quant = jnp.dot(onehot, cb, preferred_element_type=jnp.float32)
        mu = ((1.0 - AT) * (a_ref[0] + a_ref[1])
              + AT * (p_ref[0] + p_ref[1]))[:, :64]
        z = jnp.concatenate([quant, mu], axis=1)
        z_ref[...] = z
        de = jnp.dot(z, w_ref[...], preferred_element_type=jnp.float32)
        de_ref[...] = _bn_elu(de + b_ref[...], g_ref[...], bb_ref[...],
                              m_ref[...], v_ref[...])
        cl = cl_ref[...]
        dq = (jnp.sum(z * z, 1, keepdims=True)
              + jnp.sum(cl * cl, 1)[None, :]
              - 2.0 * lax.dot_general(z, cl, (((1,), (1,)), ((), ())),
                                      preferred_element_type=jnp.float32))
        q0 = 1.0 / (1.0 + dq)
        qd_ref[...] = q0 / jnp.sum(q0, axis=1, keepdims=True)

    full = lambda shape: pl.BlockSpec(shape, lambda i: (0,) * len(shape))
    fh2 = feat.shape[1]
    dmu = o2d.shape[2]
    ncl = cluster.shape[0]
    return pl.pallas_call(
        body,
        grid=(n // blk,),
        in_specs=[pl.BlockSpec((blk, fh2), lambda i: (i, 0)),
                  full((ncb, fh2)),
                  pl.BlockSpec((NC, blk, dmu), lambda i: (0, i, 0)),
                  pl.BlockSpec((NC, blk, dmu), lambda i: (0, i, 0)),
                  full((fh2 + 64, d_out)), full((d_out,)), full((d_out,)),
                  full((d_out,)), full((d_out,)), full((d_out,)),
                  full((ncl, fh2 + 64))],
        out_specs=[pl.BlockSpec((blk, fh2 + 64), lambda i: (i, 0)),
                   pl.BlockSpec((blk, d_out), lambda i: (i, 0)),
                   pl.BlockSpec((blk, ncl), lambda i: (i, 0))],
        out_shape=[jax.ShapeDtypeStruct((n, fh2 + 64), jnp.float32),
                   jax.ShapeDtypeStruct((n, d_out), jnp.float32),
                   jax.ShapeDtypeStruct((n, ncl), jnp.float32)],
    )(feat, codebook, o2d, o2e, dec_W, dec_b, bg, bb, bm, bv, cluster)


def kernel(x, adj, adj_prue, training,
           enc_W1, enc_b1, bn1_g, bn1_b, bn1_m, bn1_v,
           enc_W2, enc_b2, bn2_g, bn2_b, bn2_m, bn2_v,
           gc1_Wq, gc1_Wk, gc1_Wv, gc1_Ws, gc1_bq, gc1_bk, gc1_bv, gc1_bs,
           ch_Wq, ch_Wk, ch_Wv, ch_Ws, ch_bq, ch_bk, ch_bv, ch_bs,
           gc2_Wq, gc2_Wk, gc2_Wv, gc2_Ws, gc2_bq, gc2_bk, gc2_bv, gc2_bs,
           codebook, dec_W, dec_b, bnd_g, bnd_b, bnd_m, bnd_v, cluster):
    srcA, dstA = adj[0], adj[1]
    srcP, dstP = adj_prue[0], adj_prue[1]

    Wc1 = jnp.concatenate([gc1_Wq, gc1_Wk, gc1_Wv, gc1_Ws], axis=1)
    bc1 = jnp.concatenate([gc1_bq, gc1_bk, gc1_bv, gc1_bs])
    Wc2 = jnp.concatenate([ch_Wq, ch_Wk, ch_Wv, ch_Ws], axis=1)
    bc2 = jnp.concatenate([ch_bq, ch_bk, ch_bv, ch_bs])
    _pw = lambda W: jnp.pad(W, ((0, 0), (0, 64)))
    _pb = lambda b: jnp.pad(b, (0, 64))
    Wc3 = jnp.concatenate([_pw(gc2_Wq), _pw(gc2_Wk), _pw(gc2_Wv),
                           _pw(gc2_Ws)], axis=1)
    bc3 = jnp.concatenate([_pb(gc2_bq), _pb(gc2_bk), _pb(gc2_bv),
                           _pb(gc2_bs)])

    feat = _encoder(x, enc_W1, enc_b1, bn1_g, bn1_b, bn1_m, bn1_v,
                    enc_W2, enc_b2, bn2_g, bn2_b, bn2_m, bn2_v)

    _s128 = 1.0 / math.sqrt(128.0)
    _s64 = 1.0 / math.sqrt(64.0)

    q1, k1, v1, sk1 = _qkvs_direct(feat, Wc1, bc1, 128)
    ex1f = _ex_mat(q1, k1, _s128)
    exA, denA, aggA = _sc_ab(ex1f, v1, srcA, dstA, d=128)
    exP, denP, aggP = _sc_ab(ex1f, v1, srcP, dstP, d=128)
    h1, g1 = _comb(denA, aggA, sk1)
    h1p, g1p = _comb(denP, aggP, sk1)
    o2A = _sc_c(g1, srcA, dstA, exA, d=128)
    o2P = _sc_c(g1p, srcP, dstP, exP, d=128)

    q2, k2, v2, sk2 = _qkvs_mix(o2A, o2P, Wc2, bc2, 128)
    ex2f = _ex_mat(q2, k2, _s128)
    exB, denB, aggB = _sc_ab(ex2f, v2, srcA, dstA, d=128)
    x1, gB = _comb(denB, aggB, sk2)
    o2B = _sc_c(gB, srcA, dstA, exB, d=128)
    q3, k3, v3, sk3 = _qkvs_direct(x1, Wc2, bc2, 128)
    ex3f = _ex_mat(q3, k3, _s128)
    exC, denC, aggC = _sc_ab(ex3f, v3, srcP, dstP, d=128)
    xp, gC = _comb(denC, aggC, sk3)
    o2C = _sc_c(gC, srcP, dstP, exC, d=128)

    q4, k4, v4, sk4 = _qkvs_mix(o2B, o2C, Wc3, bc3, 128)
    ex4f = _ex_mat(q4, k4, _s64)
    exD, denD, aggD = _sc_ab(ex4f, v4, srcA, dstA, d=128)
    exE, denE, aggE = _sc_ab(ex4f, v4, srcP, dstP, d=128)
    muA, gD = _comb(denD, aggD, sk4)
    muP, gE = _comb(denE, aggE, sk4)
    o2D = _sc_c(gD, srcA, dstA, exD, d=128)
    o2E = _sc_c(gE, srcP, dstP, exE, d=128)

    z, de, qd = _decoder(feat, codebook, o2D, o2E, dec_W, dec_b,
                         bnd_g, bnd_b, bnd_m, bnd_v, cluster)
    return z, de, qd, feat
